# Initial kernel scaffold; baseline (speedup 1.0000x reference)
import functools

import jax
import jax.numpy as jnp
from jax import lax
from jax.experimental import pallas as pl
from jax.experimental.pallas import tpu as pltpu
import jax.experimental.pallas.tpu_sc as plsc

_N = 50000
_E = 800000
_H = 64
_G = 512
_C = 10

_NCORES = 2
_NSUB = 16
_HALF = _N // _NCORES
_STRIPE = 1568
_AGG_ROWS = _NSUB * _STRIPE
_CH = 80
_EPT = _E // _NSUB
_NCH = _EPT // _CH
_ZROWS = 112

_PCH = 80
_PNCH = _N // _PCH
_PK = -(-_PNCH // (_NCORES * _NSUB))


def _zero_vmem(ref, rows, width):
    def body(r, _):
        for j in range(width // 16):
            ref[r, pl.ds(16 * j, 16)] = jnp.zeros((16,), jnp.float32)
        return 0
    lax.fori_loop(0, rows, body, 0)


def _edge_body(kd_hbm, qv_hbm, e3_hbm, out_hbm,
               edge_v, idx_v, krows, qvrows, acc, zbuf, agg_sh, sem):
    c = lax.axis_index("c")
    s = lax.axis_index("s")
    base_c = c * _HALF

    _zero_vmem(zbuf, _ZROWS, _H)
    def zc(k, _):
        off = pl.multiple_of(s * _STRIPE + k * _ZROWS, 8)
        pltpu.sync_copy(zbuf, agg_sh.at[pl.ds(off, _ZROWS)])
        return 0
    lax.fori_loop(0, _STRIPE // _ZROWS, zc, 0)
    plsc.subcore_barrier()

    def chunk(ci, _):
        gc = s * _NCH + ci
        pltpu.sync_copy(e3_hbm.at[gc], edge_v)
        src_idx = edge_v.at[0]
        dst_idx = edge_v.at[1]
        cp1 = pltpu.async_copy(kd_hbm.at[dst_idx], krows, sem)
        cp2 = pltpu.async_copy(qv_hbm.at[src_idx], qvrows, sem)
        def ix(r, _):
            o = pl.multiple_of(r * 16, 16)
            d = edge_v[1, pl.ds(o, 16)]
            ld = d - base_c
            m = (ld >= 0) & (ld < _HALF)
            idx_v[pl.ds(o, 16)] = jnp.where(m, ld, _HALF)
            return 0
        lax.fori_loop(0, _CH // 16, ix, 0)
        cp1.wait()
        cp2.wait()
        def edge(e, _):
            for j in range(_H // 16):
                kj = krows[e, pl.ds(16 * j, 16)]
                qj = qvrows[e, pl.ds(16 * j, 16)]
                vj = qvrows[e, pl.ds(_H + 16 * j, 16)]
                g = 1.0 / (1.0 + jnp.exp(-(kj + qj)))
                acc[e, pl.ds(16 * j, 16)] = g * vj
            return 0
        lax.fori_loop(0, _CH, edge, 0)
        pltpu.sync_copy(acc, agg_sh.at[idx_v], add=True)
        return 0
    lax.fori_loop(0, _NCH, chunk, 0)
    plsc.subcore_barrier()

    tail = _HALF - (_NSUB - 1) * _STRIPE
    so = pl.multiple_of(s * _STRIPE, 8)
    pltpu.sync_copy(agg_sh.at[pl.ds(so, tail)],
                    out_hbm.at[pl.ds(base_c + so, tail)])
    @pl.when(s < _NSUB - 1)
    def _():
        o2 = pl.multiple_of(s * _STRIPE + tail, 8)
        pltpu.sync_copy(agg_sh.at[pl.ds(o2, _STRIPE - tail)],
                        out_hbm.at[pl.ds(base_c + o2, _STRIPE - tail)])


_edge_call = functools.partial(
    pl.kernel,
    _edge_body,
    out_type=jax.ShapeDtypeStruct((_N, _H), jnp.float32),
    mesh=plsc.VectorSubcoreMesh(core_axis_name="c", subcore_axis_name="s"),
    scratch_types=[
        pltpu.VMEM((2, _CH), jnp.int32),
        pltpu.VMEM((_CH,), jnp.int32),
        pltpu.VMEM((_CH, _H), jnp.float32),
        pltpu.VMEM((_CH, 2 * _H), jnp.float32),
        pltpu.VMEM((_CH, _H), jnp.float32),
        pltpu.VMEM((_ZROWS, _H), jnp.float32),
        pltpu.VMEM_SHARED((_AGG_ROWS, _H), jnp.float32),
        pltpu.SemaphoreType.DMA,
    ],
)


def _pool_body(agg_hbm, sk_hbm, batch_hbm, sums_hbm, cnts_hbm,
               b_v, rows_a, rows_s, ones_v, zbuf, sums_sh, cnts_sh):
    c = lax.axis_index("c")
    s = lax.axis_index("s")
    wid = c * _NSUB + s

    _zero_vmem(zbuf, 32, _H)
    def fill_ones(r, _):
        ones_v[r, pl.ds(0, 16)] = jnp.full((16,), 1.0, jnp.float32)
        return 0
    lax.fori_loop(0, _PCH, fill_ones, 0)
    so = pl.multiple_of(s * 32, 8)
    pltpu.sync_copy(zbuf, sums_sh.at[pl.ds(so, 32)])
    pltpu.sync_copy(zbuf.at[pl.ds(0, 32), pl.ds(0, 16)], cnts_sh.at[pl.ds(so, 32)])
    plsc.subcore_barrier()

    def step(k, _):
        ch = k * (_NCORES * _NSUB) + wid
        @pl.when(ch < _PNCH)
        def _():
            off = pl.multiple_of(ch * _PCH, 8)
            pltpu.sync_copy(agg_hbm.at[pl.ds(off, _PCH)], rows_a)
            pltpu.sync_copy(sk_hbm.at[pl.ds(off, _PCH)], rows_s)
            pltpu.sync_copy(batch_hbm.at[pl.ds(off, _PCH)], b_v)
            def addrow(e, _):
                for j in range(_H // 16):
                    rows_a[e, pl.ds(16 * j, 16)] = (
                        rows_a[e, pl.ds(16 * j, 16)] + rows_s[e, pl.ds(16 * j, 16)])
                return 0
            lax.fori_loop(0, _PCH, addrow, 0)
            pltpu.sync_copy(rows_a, sums_sh.at[b_v], add=True)
            pltpu.sync_copy(ones_v, cnts_sh.at[b_v], add=True)
        return 0
    lax.fori_loop(0, _PK, step, 0)
    plsc.subcore_barrier()

    @pl.when(s == 0)
    def _():
        pltpu.sync_copy(sums_sh, sums_hbm.at[c])
        pltpu.sync_copy(cnts_sh, cnts_hbm.at[c])


_pool_call = functools.partial(
    pl.kernel,
    _pool_body,
    out_type=(jax.ShapeDtypeStruct((_NCORES, _G, _H), jnp.float32),
              jax.ShapeDtypeStruct((_NCORES, _G, 16), jnp.float32)),
    mesh=plsc.VectorSubcoreMesh(core_axis_name="c", subcore_axis_name="s"),
    scratch_types=[
        pltpu.VMEM((_PCH,), jnp.int32),
        pltpu.VMEM((_PCH, _H), jnp.float32),
        pltpu.VMEM((_PCH, _H), jnp.float32),
        pltpu.VMEM((_PCH, 16), jnp.float32),
        pltpu.VMEM((32, _H), jnp.float32),
        pltpu.VMEM_SHARED((_G, _H), jnp.float32),
        pltpu.VMEM_SHARED((_G, 16), jnp.float32),
    ],
)


_RB = 400
_NB = _N // _RB


def _dense1_body(x_ref, wk, bk, wqv, bqv, ws, bs, kd, qv, sk):
    X = x_ref[...]
    kd[...] = X * wk[...] + bk[...]
    qv[...] = X * wqv[...] + bqv[...]
    sk[...] = X * ws[...] + bs[...]


def _dense2_body(agg_ref, skin_ref, wk, bk, wqv, bqv, ws, bs, kd, qv, sk):
    X = jnp.maximum(agg_ref[...] + skin_ref[...], 0.0)
    kd[...] = jnp.dot(X, wk[...], preferred_element_type=jnp.float32) + bk[...]
    qv[...] = jnp.dot(X, wqv[...], preferred_element_type=jnp.float32) + bqv[...]
    sk[...] = jnp.dot(X, ws[...], preferred_element_type=jnp.float32) + bs[...]


def _wspec(din, dout):
    return pl.BlockSpec((din, dout), lambda i: (0, 0))


def _nspec(width):
    return pl.BlockSpec((_RB, width), lambda i: (i, 0))


def _dense_out_types():
    return (jax.ShapeDtypeStruct((_N, _H), jnp.float32),
            jax.ShapeDtypeStruct((_N, 2 * _H), jnp.float32),
            jax.ShapeDtypeStruct((_N, _H), jnp.float32))


def _dense1(x, wk, bk, wqv, bqv, ws, bs):
    return pl.pallas_call(
        _dense1_body,
        grid=(_NB,),
        in_specs=[_nspec(1), _wspec(1, _H), _wspec(1, _H), _wspec(1, 2 * _H),
                  _wspec(1, 2 * _H), _wspec(1, _H), _wspec(1, _H)],
        out_specs=[_nspec(_H), _nspec(2 * _H), _nspec(_H)],
        out_shape=_dense_out_types(),
    )(x, wk, bk, wqv, bqv, ws, bs)


def _dense2(agg, skin, wk, bk, wqv, bqv, ws, bs):
    return pl.pallas_call(
        _dense2_body,
        grid=(_NB,),
        in_specs=[_nspec(_H), _nspec(_H), _wspec(_H, _H), _wspec(1, _H),
                  _wspec(_H, 2 * _H), _wspec(1, 2 * _H), _wspec(_H, _H),
                  _wspec(1, _H)],
        out_specs=[_nspec(_H), _nspec(2 * _H), _nspec(_H)],
        out_shape=_dense_out_types(),
    )(agg, skin, wk, bk, wqv, bqv, ws, bs)


def _final_body(sums_ref, cnts_ref, w, b, out_ref):
    sm = sums_ref[0] + sums_ref[1]
    cn = cnts_ref[0] + cnts_ref[1]
    cnt = cn[:, 0:1]
    pooled = sm / jnp.maximum(cnt, 1.0)
    out_ref[...] = jnp.dot(pooled, w[...], preferred_element_type=jnp.float32) + b[...]


def _final(sums, cnts, w, b):
    return pl.pallas_call(
        _final_body,
        grid=(1,),
        in_specs=[pl.BlockSpec((_NCORES, _G, _H), lambda i: (0, 0, 0)),
                  pl.BlockSpec((_NCORES, _G, 16), lambda i: (0, 0, 0)),
                  _wspec(_H, _C), _wspec(1, _C)],
        out_specs=pl.BlockSpec((_G, _C), lambda i: (0, 0)),
        out_shape=jax.ShapeDtypeStruct((_G, _C), jnp.float32),
    )(sums, cnts, w, b)


def _conv_weights(p):
    wk = p["key"]["W"]
    bk = p["key"]["b"].reshape(1, -1)
    wqv = jnp.concatenate([p["query"]["W"], p["value"]["W"]], axis=1)
    bqv = jnp.concatenate([p["query"]["b"], p["value"]["b"]]).reshape(1, -1)
    ws = p["skip"]["W"]
    bs = p["skip"]["b"].reshape(1, -1)
    return wk, bk, wqv, bqv, ws, bs


def kernel(x, edge_index, batch, params):
    e3 = edge_index.reshape(2, _E // _CH, _CH).transpose(1, 0, 2)

    kd, qv, sk = _dense1(x, *_conv_weights(params["conv1"]))
    agg = _edge_call()(kd, qv, e3)
    for name in ("conv2", "conv2_1", "conv3"):
        kd, qv, sk = _dense2(agg, sk, *_conv_weights(params[name]))
        agg = _edge_call()(kd, qv, e3)

    sums, cnts = _pool_call()(agg, sk, batch)
    return _final(sums, cnts, params["lin"]["W"], params["lin"]["b"].reshape(1, -1))


# speedup vs baseline: 1.2447x; 1.2447x over previous
import functools

import jax
import jax.numpy as jnp
from jax import lax
from jax.experimental import pallas as pl
from jax.experimental.pallas import tpu as pltpu
import jax.experimental.pallas.tpu_sc as plsc

_N = 50000
_E = 800000
_H = 64
_G = 512
_C = 10

_NCORES = 2
_NSUB = 16
_HALF = _N // _NCORES
_STRIPE = 1568
_AGG_ROWS = _NSUB * _STRIPE
_CH = 80
_EPT = _E // _NSUB
_NCH = _EPT // _CH

_PCH = 80
_PNCH = _N // _PCH
_PK = -(-_PNCH // (_NCORES * _NSUB))

_SC_PARAMS = pltpu.CompilerParams(use_tc_tiling_on_sc=False)


def _zero_vmem(ref, rows, width):
    def body(r, _):
        for j in range(width // 16):
            ref[r, pl.ds(16 * j, 16)] = jnp.zeros((16,), jnp.float32)
        return 0
    lax.fori_loop(0, rows, body, 0)


def _edge_body(kd_hbm, qv_hbm, e3_hbm, out_hbm,
               edge_v, idx_v, krows, qvrows, acc, agg_sh, sem):
    c = lax.axis_index("c")
    s = lax.axis_index("s")
    base_c = c * _HALF

    _zero_vmem(acc, _CH, _H)
    def zc(k, _):
        off = pl.multiple_of(s * _STRIPE + k * _CH, 8)
        pltpu.sync_copy(acc, agg_sh.at[pl.ds(off, _CH)])
        return 0
    lax.fori_loop(0, _STRIPE // _CH, zc, 0)
    zo = pl.multiple_of(s * _STRIPE + (_STRIPE // _CH) * _CH, 8)
    pltpu.sync_copy(acc.at[pl.ds(0, _STRIPE - (_STRIPE // _CH) * _CH)],
                    agg_sh.at[pl.ds(zo, _STRIPE - (_STRIPE // _CH) * _CH)])
    plsc.subcore_barrier()

    def chunk(ci, _):
        gc = s * _NCH + ci
        pltpu.sync_copy(e3_hbm.at[gc], edge_v)
        src_idx = edge_v.at[0]
        dst_idx = edge_v.at[1]
        cp1 = pltpu.async_copy(kd_hbm.at[dst_idx], krows, sem)
        cp2 = pltpu.async_copy(qv_hbm.at[src_idx], qvrows, sem)
        def ix(r, _):
            o = pl.multiple_of(r * 16, 16)
            d = edge_v[1, pl.ds(o, 16)]
            ld = d - base_c
            m = (ld >= 0) & (ld < _HALF)
            idx_v[pl.ds(o, 16)] = jnp.where(m, ld, _HALF)
            return 0
        lax.fori_loop(0, _CH // 16, ix, 0)
        cp1.wait()
        cp2.wait()
        def edge(e, _):
            for j in range(_H // 16):
                kj = krows[e, pl.ds(16 * j, 16)]
                qj = qvrows[e, pl.ds(16 * j, 16)]
                vj = qvrows[e, pl.ds(_H + 16 * j, 16)]
                g = 1.0 / (1.0 + jnp.exp(-(kj + qj)))
                acc[e, pl.ds(16 * j, 16)] = g * vj
            return 0
        lax.fori_loop(0, _CH, edge, 0)
        pltpu.sync_copy(acc, agg_sh.at[idx_v], add=True)
        return 0
    lax.fori_loop(0, _NCH, chunk, 0)
    plsc.subcore_barrier()

    tail = _HALF - (_NSUB - 1) * _STRIPE
    so = pl.multiple_of(s * _STRIPE, 8)
    pltpu.sync_copy(agg_sh.at[pl.ds(so, tail)],
                    out_hbm.at[pl.ds(base_c + so, tail)])
    @pl.when(s < _NSUB - 1)
    def _():
        o2 = pl.multiple_of(s * _STRIPE + tail, 8)
        pltpu.sync_copy(agg_sh.at[pl.ds(o2, _STRIPE - tail)],
                        out_hbm.at[pl.ds(base_c + o2, _STRIPE - tail)])


_edge_call = functools.partial(
    pl.kernel,
    _edge_body,
    out_type=jax.ShapeDtypeStruct((_N, _H), jnp.float32),
    mesh=plsc.VectorSubcoreMesh(core_axis_name="c", subcore_axis_name="s"),
    scratch_types=[
        pltpu.VMEM((2, _CH), jnp.int32),
        pltpu.VMEM((_CH,), jnp.int32),
        pltpu.VMEM((_CH, _H), jnp.float32),
        pltpu.VMEM((_CH, 2 * _H), jnp.float32),
        pltpu.VMEM((_CH, _H), jnp.float32),
        pltpu.VMEM_SHARED((_AGG_ROWS, _H), jnp.float32),
        pltpu.SemaphoreType.DMA,
    ],
    compiler_params=_SC_PARAMS,
)


def _pool_body(agg_hbm, sk_hbm, batch_hbm, sums_hbm, cnts_hbm,
               b_v, rows_a, rows_s, ones_v, zbuf2, sums_sh, cnts_sh):
    c = lax.axis_index("c")
    s = lax.axis_index("s")
    wid = c * _NSUB + s

    _zero_vmem(rows_a, 32, _H)
    _zero_vmem(zbuf2, 32, 16)
    def fill_ones(r, _):
        ones_v[r, pl.ds(0, 16)] = jnp.full((16,), 1.0, jnp.float32)
        return 0
    lax.fori_loop(0, _PCH, fill_ones, 0)
    so = pl.multiple_of(s * 32, 8)
    pltpu.sync_copy(rows_a.at[pl.ds(0, 32)], sums_sh.at[pl.ds(so, 32)])
    pltpu.sync_copy(zbuf2, cnts_sh.at[pl.ds(so, 32)])
    plsc.subcore_barrier()

    def step(k, _):
        ch = k * (_NCORES * _NSUB) + wid
        @pl.when(ch < _PNCH)
        def _():
            off = pl.multiple_of(ch * _PCH, 8)
            pltpu.sync_copy(agg_hbm.at[pl.ds(off, _PCH)], rows_a)
            pltpu.sync_copy(sk_hbm.at[pl.ds(off, _PCH)], rows_s)
            pltpu.sync_copy(batch_hbm.at[pl.ds(off, _PCH)], b_v)
            def addrow(e, _):
                for j in range(_H // 16):
                    rows_a[e, pl.ds(16 * j, 16)] = (
                        rows_a[e, pl.ds(16 * j, 16)] + rows_s[e, pl.ds(16 * j, 16)])
                return 0
            lax.fori_loop(0, _PCH, addrow, 0)
            pltpu.sync_copy(rows_a, sums_sh.at[b_v], add=True)
            pltpu.sync_copy(ones_v, cnts_sh.at[b_v], add=True)
        return 0
    lax.fori_loop(0, _PK, step, 0)
    plsc.subcore_barrier()

    @pl.when(s == 0)
    def _():
        pltpu.sync_copy(sums_sh, sums_hbm.at[c])
        pltpu.sync_copy(cnts_sh, cnts_hbm.at[c])


_pool_call = functools.partial(
    pl.kernel,
    _pool_body,
    out_type=(jax.ShapeDtypeStruct((_NCORES, _G, _H), jnp.float32),
              jax.ShapeDtypeStruct((_NCORES, _G, 16), jnp.float32)),
    mesh=plsc.VectorSubcoreMesh(core_axis_name="c", subcore_axis_name="s"),
    scratch_types=[
        pltpu.VMEM((_PCH,), jnp.int32),
        pltpu.VMEM((_PCH, _H), jnp.float32),
        pltpu.VMEM((_PCH, _H), jnp.float32),
        pltpu.VMEM((_PCH, 16), jnp.float32),
        pltpu.VMEM((32, 16), jnp.float32),
        pltpu.VMEM_SHARED((_G, _H), jnp.float32),
        pltpu.VMEM_SHARED((_G, 16), jnp.float32),
    ],
    compiler_params=_SC_PARAMS,
)


_RB = 400
_NB = _N // _RB


def _dense1_body(x_ref, wk, bk, wqv, bqv, ws, bs, kd, qv, sk):
    X = x_ref[...]
    kd[...] = X * wk[...] + bk[...]
    qv[...] = X * wqv[...] + bqv[...]
    sk[...] = X * ws[...] + bs[...]


def _dense2_body(agg_ref, skin_ref, wk, bk, wqv, bqv, ws, bs, kd, qv, sk):
    X = jnp.maximum(agg_ref[...] + skin_ref[...], 0.0)
    kd[...] = jnp.dot(X, wk[...], preferred_element_type=jnp.float32) + bk[...]
    qv[...] = jnp.dot(X, wqv[...], preferred_element_type=jnp.float32) + bqv[...]
    sk[...] = jnp.dot(X, ws[...], preferred_element_type=jnp.float32) + bs[...]


def _wspec(din, dout):
    return pl.BlockSpec((din, dout), lambda i: (0, 0))


def _nspec(width):
    return pl.BlockSpec((_RB, width), lambda i: (i, 0))


def _dense_out_types():
    return (jax.ShapeDtypeStruct((_N, _H), jnp.float32),
            jax.ShapeDtypeStruct((_N, 2 * _H), jnp.float32),
            jax.ShapeDtypeStruct((_N, _H), jnp.float32))


def _dense1(x, wk, bk, wqv, bqv, ws, bs):
    return pl.pallas_call(
        _dense1_body,
        grid=(_NB,),
        in_specs=[_nspec(1), _wspec(1, _H), _wspec(1, _H), _wspec(1, 2 * _H),
                  _wspec(1, 2 * _H), _wspec(1, _H), _wspec(1, _H)],
        out_specs=[_nspec(_H), _nspec(2 * _H), _nspec(_H)],
        out_shape=_dense_out_types(),
    )(x, wk, bk, wqv, bqv, ws, bs)


def _dense2(agg, skin, wk, bk, wqv, bqv, ws, bs):
    return pl.pallas_call(
        _dense2_body,
        grid=(_NB,),
        in_specs=[_nspec(_H), _nspec(_H), _wspec(_H, _H), _wspec(1, _H),
                  _wspec(_H, 2 * _H), _wspec(1, 2 * _H), _wspec(_H, _H),
                  _wspec(1, _H)],
        out_specs=[_nspec(_H), _nspec(2 * _H), _nspec(_H)],
        out_shape=_dense_out_types(),
    )(agg, skin, wk, bk, wqv, bqv, ws, bs)


def _final_body(sums_ref, cnts_ref, w, b, out_ref):
    sm = sums_ref[0] + sums_ref[1]
    cn = cnts_ref[0] + cnts_ref[1]
    cnt = cn[:, 0:1]
    pooled = sm / jnp.maximum(cnt, 1.0)
    out_ref[...] = jnp.dot(pooled, w[...], preferred_element_type=jnp.float32) + b[...]


def _final(sums, cnts, w, b):
    return pl.pallas_call(
        _final_body,
        grid=(1,),
        in_specs=[pl.BlockSpec((_NCORES, _G, _H), lambda i: (0, 0, 0)),
                  pl.BlockSpec((_NCORES, _G, 16), lambda i: (0, 0, 0)),
                  _wspec(_H, _C), _wspec(1, _C)],
        out_specs=pl.BlockSpec((_G, _C), lambda i: (0, 0)),
        out_shape=jax.ShapeDtypeStruct((_G, _C), jnp.float32),
    )(sums, cnts, w, b)


def _conv_weights(p):
    wk = p["key"]["W"]
    bk = p["key"]["b"].reshape(1, -1)
    wqv = jnp.concatenate([p["query"]["W"], p["value"]["W"]], axis=1)
    bqv = jnp.concatenate([p["query"]["b"], p["value"]["b"]]).reshape(1, -1)
    ws = p["skip"]["W"]
    bs = p["skip"]["b"].reshape(1, -1)
    return wk, bk, wqv, bqv, ws, bs


def kernel(x, edge_index, batch, params):
    e3 = edge_index.reshape(2, _E // _CH, _CH).transpose(1, 0, 2)

    kd, qv, sk = _dense1(x, *_conv_weights(params["conv1"]))
    agg = _edge_call()(kd, qv, e3)
    for name in ("conv2", "conv2_1", "conv3"):
        kd, qv, sk = _dense2(agg, sk, *_conv_weights(params[name]))
        agg = _edge_call()(kd, qv, e3)

    sums, cnts = _pool_call()(agg, sk, batch)
    return _final(sums, cnts, params["lin"]["W"], params["lin"]["b"].reshape(1, -1))


# speedup vs baseline: 3.7538x; 3.0158x over previous
import functools

import jax
import jax.numpy as jnp
from jax import lax
from jax.experimental import pallas as pl
from jax.experimental.pallas import tpu as pltpu
import jax.experimental.pallas.tpu_sc as plsc

_N = 50000
_E = 800000
_H = 64
_G = 512
_C = 10

_NCORES = 2
_NSUB = 16
_HALF = _N // _NCORES
_STRIPE = 1568
_AGG_ROWS = _NSUB * _STRIPE
_CH = 80
_EPT = _E // _NSUB
_NCH = _EPT // _CH

_PCH = 80
_PNCH = _N // _PCH
_PK = -(-_PNCH // (_NCORES * _NSUB))

_SC_PARAMS = pltpu.CompilerParams(use_tc_tiling_on_sc=False)


def _zero_vmem(ref, rows, width):
    def body(r, _):
        for j in range(width // 16):
            ref[r, pl.ds(16 * j, 16)] = jnp.zeros((16,), jnp.float32)
        return 0
    lax.fori_loop(0, rows, body, 0)


def _edge_body(kd_hbm, qv_hbm, e3_hbm, out_hbm,
               edge_v, idx_v, krows, qvrows, acc, agg_sh, sem):
    c = lax.axis_index("c")
    s = lax.axis_index("s")
    base_c = c * _HALF

    _zero_vmem(acc, _CH, _H)
    def zc(k, _):
        off = pl.multiple_of(s * _STRIPE + k * _CH, 8)
        pltpu.sync_copy(acc, agg_sh.at[pl.ds(off, _CH)])
        return 0
    lax.fori_loop(0, _STRIPE // _CH, zc, 0)
    zo = pl.multiple_of(s * _STRIPE + (_STRIPE // _CH) * _CH, 8)
    pltpu.sync_copy(acc.at[pl.ds(0, _STRIPE - (_STRIPE // _CH) * _CH)],
                    agg_sh.at[pl.ds(zo, _STRIPE - (_STRIPE // _CH) * _CH)])
    plsc.subcore_barrier()

    def chunk(ci, _):
        gc = s * _NCH + ci
        pltpu.sync_copy(e3_hbm.at[gc], edge_v)
        src_idx = edge_v.at[0]
        dst_idx = edge_v.at[1]
        cp1 = pltpu.async_copy(kd_hbm.at[dst_idx], krows, sem)
        cp2 = pltpu.async_copy(qv_hbm.at[src_idx], qvrows, sem)
        def ix(r, _):
            o = pl.multiple_of(r * 16, 16)
            d = edge_v[1, pl.ds(o, 16)]
            ld = d - base_c
            m = (ld >= 0) & (ld < _HALF)
            idx_v[pl.ds(o, 16)] = jnp.where(m, ld, _HALF)
            return 0
        lax.fori_loop(0, _CH // 16, ix, 0)
        cp1.wait()
        cp2.wait()
        @plsc.parallel_loop(0, _CH, unroll=4)
        def _(e):
            for j in range(_H // 16):
                kj = krows[e, pl.ds(16 * j, 16)]
                qj = qvrows[e, pl.ds(16 * j, 16)]
                vj = qvrows[e, pl.ds(_H + 16 * j, 16)]
                g = 1.0 / (1.0 + jnp.exp(-(kj + qj)))
                acc[e, pl.ds(16 * j, 16)] = g * vj
        pltpu.sync_copy(acc, agg_sh.at[idx_v], add=True)
        return 0
    lax.fori_loop(0, _NCH, chunk, 0)
    plsc.subcore_barrier()

    tail = _HALF - (_NSUB - 1) * _STRIPE
    so = pl.multiple_of(s * _STRIPE, 8)
    pltpu.sync_copy(agg_sh.at[pl.ds(so, tail)],
                    out_hbm.at[pl.ds(base_c + so, tail)])
    @pl.when(s < _NSUB - 1)
    def _():
        o2 = pl.multiple_of(s * _STRIPE + tail, 8)
        pltpu.sync_copy(agg_sh.at[pl.ds(o2, _STRIPE - tail)],
                        out_hbm.at[pl.ds(base_c + o2, _STRIPE - tail)])


_edge_call = functools.partial(
    pl.kernel,
    _edge_body,
    out_type=jax.ShapeDtypeStruct((_N, _H), jnp.float32),
    mesh=plsc.VectorSubcoreMesh(core_axis_name="c", subcore_axis_name="s"),
    scratch_types=[
        pltpu.VMEM((2, _CH), jnp.int32),
        pltpu.VMEM((_CH,), jnp.int32),
        pltpu.VMEM((_CH, _H), jnp.float32),
        pltpu.VMEM((_CH, 2 * _H), jnp.float32),
        pltpu.VMEM((_CH, _H), jnp.float32),
        pltpu.VMEM_SHARED((_AGG_ROWS, _H), jnp.float32),
        pltpu.SemaphoreType.DMA,
    ],
    compiler_params=_SC_PARAMS,
)


def _pool_body(agg_hbm, sk_hbm, batch_hbm, sums_hbm, cnts_hbm,
               b_v, rows_a, rows_s, ones_v, zbuf2, sums_sh, cnts_sh):
    c = lax.axis_index("c")
    s = lax.axis_index("s")
    wid = c * _NSUB + s

    _zero_vmem(rows_a, 32, _H)
    _zero_vmem(zbuf2, 32, 16)
    def fill_ones(r, _):
        ones_v[r, pl.ds(0, 16)] = jnp.full((16,), 1.0, jnp.float32)
        return 0
    lax.fori_loop(0, _PCH, fill_ones, 0)
    so = pl.multiple_of(s * 32, 8)
    pltpu.sync_copy(rows_a.at[pl.ds(0, 32)], sums_sh.at[pl.ds(so, 32)])
    pltpu.sync_copy(zbuf2, cnts_sh.at[pl.ds(so, 32)])
    plsc.subcore_barrier()

    def step(k, _):
        ch = k * (_NCORES * _NSUB) + wid
        @pl.when(ch < _PNCH)
        def _():
            off = pl.multiple_of(ch * _PCH, 8)
            pltpu.sync_copy(agg_hbm.at[pl.ds(off, _PCH)], rows_a)
            pltpu.sync_copy(sk_hbm.at[pl.ds(off, _PCH)], rows_s)
            pltpu.sync_copy(batch_hbm.at[pl.ds(off, _PCH)], b_v)
            def addrow(e, _):
                for j in range(_H // 16):
                    rows_a[e, pl.ds(16 * j, 16)] = (
                        rows_a[e, pl.ds(16 * j, 16)] + rows_s[e, pl.ds(16 * j, 16)])
                return 0
            lax.fori_loop(0, _PCH, addrow, 0)
            pltpu.sync_copy(rows_a, sums_sh.at[b_v], add=True)
            pltpu.sync_copy(ones_v, cnts_sh.at[b_v], add=True)
        return 0
    lax.fori_loop(0, _PK, step, 0)
    plsc.subcore_barrier()

    @pl.when(s == 0)
    def _():
        pltpu.sync_copy(sums_sh, sums_hbm.at[c])
        pltpu.sync_copy(cnts_sh, cnts_hbm.at[c])


_pool_call = functools.partial(
    pl.kernel,
    _pool_body,
    out_type=(jax.ShapeDtypeStruct((_NCORES, _G, _H), jnp.float32),
              jax.ShapeDtypeStruct((_NCORES, _G, 16), jnp.float32)),
    mesh=plsc.VectorSubcoreMesh(core_axis_name="c", subcore_axis_name="s"),
    scratch_types=[
        pltpu.VMEM((_PCH,), jnp.int32),
        pltpu.VMEM((_PCH, _H), jnp.float32),
        pltpu.VMEM((_PCH, _H), jnp.float32),
        pltpu.VMEM((_PCH, 16), jnp.float32),
        pltpu.VMEM((32, 16), jnp.float32),
        pltpu.VMEM_SHARED((_G, _H), jnp.float32),
        pltpu.VMEM_SHARED((_G, 16), jnp.float32),
    ],
    compiler_params=_SC_PARAMS,
)


_RB = 400
_NB = _N // _RB


def _dense1_body(x_ref, wk, bk, wqv, bqv, ws, bs, kd, qv, sk):
    X = x_ref[...]
    kd[...] = X * wk[...] + bk[...]
    qv[...] = X * wqv[...] + bqv[...]
    sk[...] = X * ws[...] + bs[...]


def _dense2_body(agg_ref, skin_ref, wk, bk, wqv, bqv, ws, bs, kd, qv, sk):
    X = jnp.maximum(agg_ref[...] + skin_ref[...], 0.0)
    kd[...] = jnp.dot(X, wk[...], preferred_element_type=jnp.float32) + bk[...]
    qv[...] = jnp.dot(X, wqv[...], preferred_element_type=jnp.float32) + bqv[...]
    sk[...] = jnp.dot(X, ws[...], preferred_element_type=jnp.float32) + bs[...]


def _wspec(din, dout):
    return pl.BlockSpec((din, dout), lambda i: (0, 0))


def _nspec(width):
    return pl.BlockSpec((_RB, width), lambda i: (i, 0))


def _dense_out_types():
    return (jax.ShapeDtypeStruct((_N, _H), jnp.float32),
            jax.ShapeDtypeStruct((_N, 2 * _H), jnp.float32),
            jax.ShapeDtypeStruct((_N, _H), jnp.float32))


def _dense1(x, wk, bk, wqv, bqv, ws, bs):
    return pl.pallas_call(
        _dense1_body,
        grid=(_NB,),
        in_specs=[_nspec(1), _wspec(1, _H), _wspec(1, _H), _wspec(1, 2 * _H),
                  _wspec(1, 2 * _H), _wspec(1, _H), _wspec(1, _H)],
        out_specs=[_nspec(_H), _nspec(2 * _H), _nspec(_H)],
        out_shape=_dense_out_types(),
    )(x, wk, bk, wqv, bqv, ws, bs)


def _dense2(agg, skin, wk, bk, wqv, bqv, ws, bs):
    return pl.pallas_call(
        _dense2_body,
        grid=(_NB,),
        in_specs=[_nspec(_H), _nspec(_H), _wspec(_H, _H), _wspec(1, _H),
                  _wspec(_H, 2 * _H), _wspec(1, 2 * _H), _wspec(_H, _H),
                  _wspec(1, _H)],
        out_specs=[_nspec(_H), _nspec(2 * _H), _nspec(_H)],
        out_shape=_dense_out_types(),
    )(agg, skin, wk, bk, wqv, bqv, ws, bs)


def _final_body(sums_ref, cnts_ref, w, b, out_ref):
    sm = sums_ref[0] + sums_ref[1]
    cn = cnts_ref[0] + cnts_ref[1]
    cnt = cn[:, 0:1]
    pooled = sm / jnp.maximum(cnt, 1.0)
    out_ref[...] = jnp.dot(pooled, w[...], preferred_element_type=jnp.float32) + b[...]


def _final(sums, cnts, w, b):
    return pl.pallas_call(
        _final_body,
        grid=(1,),
        in_specs=[pl.BlockSpec((_NCORES, _G, _H), lambda i: (0, 0, 0)),
                  pl.BlockSpec((_NCORES, _G, 16), lambda i: (0, 0, 0)),
                  _wspec(_H, _C), _wspec(1, _C)],
        out_specs=pl.BlockSpec((_G, _C), lambda i: (0, 0)),
        out_shape=jax.ShapeDtypeStruct((_G, _C), jnp.float32),
    )(sums, cnts, w, b)


def _conv_weights(p):
    wk = p["key"]["W"]
    bk = p["key"]["b"].reshape(1, -1)
    wqv = jnp.concatenate([p["query"]["W"], p["value"]["W"]], axis=1)
    bqv = jnp.concatenate([p["query"]["b"], p["value"]["b"]]).reshape(1, -1)
    ws = p["skip"]["W"]
    bs = p["skip"]["b"].reshape(1, -1)
    return wk, bk, wqv, bqv, ws, bs


def kernel(x, edge_index, batch, params):
    e3 = edge_index.reshape(2, _E // _CH, _CH).transpose(1, 0, 2)

    kd, qv, sk = _dense1(x, *_conv_weights(params["conv1"]))
    agg = _edge_call()(kd, qv, e3)
    for name in ("conv2", "conv2_1", "conv3"):
        kd, qv, sk = _dense2(agg, sk, *_conv_weights(params[name]))
        agg = _edge_call()(kd, qv, e3)

    sums, cnts = _pool_call()(agg, sk, batch)
    return _final(sums, cnts, params["lin"]["W"], params["lin"]["b"].reshape(1, -1))


# speedup vs baseline: 5.1355x; 1.3681x over previous
import functools

import jax
import jax.numpy as jnp
from jax import lax
from jax.experimental import pallas as pl
from jax.experimental.pallas import tpu as pltpu
import jax.experimental.pallas.tpu_sc as plsc

_N = 50000
_E = 800000
_H = 64
_G = 512
_C = 10

_NCORES = 2
_NSUB = 16
_HALF = _N // _NCORES
_STRIPE = 1568
_AGG_ROWS = _NSUB * _STRIPE
_CH = 64
_E2 = 800256
_SCH = 48
_SSEG = _E2 // (_NCORES * _NSUB * _SCH)
_CAP = 25088

_PCH = 80
_PNCH = _N // _PCH
_PK = -(-_PNCH // (_NCORES * _NSUB))

_SC_PARAMS = pltpu.CompilerParams(use_tc_tiling_on_sc=False)
_SC_PARAMS_NL = pltpu.CompilerParams(use_tc_tiling_on_sc=False,
                                     needs_layout_passes=False)


def _zero_vmem(ref, rows, width):
    def body(r, _):
        for j in range(width // 16):
            ref[r, pl.ds(16 * j, 16)] = jnp.zeros((16,), jnp.float32)
        return 0
    lax.fori_loop(0, rows, body, 0)


def _part_body(e3_hbm, srcp_hbm, dstp_hbm, cnts_hbm,
               chunk_v, b0s, b0d, b1s, b1d, cntv):
    c = lax.axis_index("c")
    s = lax.axis_index("s")
    t = c * _NSUB + s
    lane16 = lax.broadcasted_iota(jnp.int32, (16,), 0)

    def scan_chunk(ci, carry):
        pltpu.sync_copy(e3_hbm.at[t * _SSEG + ci], chunk_v)
        c0, c1 = carry
        for g in range(_SCH // 16):
            sv = chunk_v[0, pl.ds(16 * g, 16)]
            dv = chunk_v[1, pl.ds(16 * g, 16)]
            m0 = dv < _HALF
            mi = jnp.where(m0, 1, 0)
            pos0 = c0 + plsc.cumsum(mi) - 1
            pos1 = c1 + plsc.cumsum(1 - mi) - 1
            plsc.store_scatter(b0s, [pos0], sv, mask=m0)
            plsc.store_scatter(b0d, [pos0], dv, mask=m0)
            m1 = jnp.logical_not(m0)
            plsc.store_scatter(b1s, [pos1], sv, mask=m1)
            plsc.store_scatter(b1d, [pos1], dv, mask=m1)
            n0 = jnp.sum(mi)
            c0 = c0 + n0
            c1 = c1 + (16 - n0)
        return (c0, c1)
    cnt0, cnt1 = lax.fori_loop(0, _SSEG, scan_chunk,
                               (jnp.int32(0), jnp.int32(0)))

    zero16 = jnp.zeros((16,), jnp.int32)
    garb16 = jnp.full((16,), _N, jnp.int32)
    for k in range(_CH // 16):
        plsc.store_scatter(b0s, [cnt0 + lane16 + 16 * k], zero16)
        plsc.store_scatter(b0d, [cnt0 + lane16 + 16 * k], garb16)
        plsc.store_scatter(b1s, [cnt1 + lane16 + 16 * k], zero16)
        plsc.store_scatter(b1d, [cnt1 + lane16 + 16 * k], garb16)
    c0p = ((cnt0 + _CH - 1) // _CH) * _CH
    c1p = ((cnt1 + _CH - 1) // _CH) * _CH

    pltpu.sync_copy(b0s, srcp_hbm.at[0, t])
    pltpu.sync_copy(b0d, dstp_hbm.at[0, t])
    pltpu.sync_copy(b1s, srcp_hbm.at[1, t])
    pltpu.sync_copy(b1d, dstp_hbm.at[1, t])
    cntv[pl.ds(0, 16)] = jnp.where(lane16 == 0, c0p // _CH,
                                   jnp.where(lane16 == 1, c1p // _CH, 0))
    pltpu.sync_copy(cntv, cnts_hbm.at[t])


_part_call = functools.partial(
    pl.kernel,
    _part_body,
    out_type=(jax.ShapeDtypeStruct((_NCORES, _NCORES * _NSUB, _CAP), jnp.int32),
              jax.ShapeDtypeStruct((_NCORES, _NCORES * _NSUB, _CAP), jnp.int32),
              jax.ShapeDtypeStruct((_NCORES * _NSUB, 16), jnp.int32)),
    mesh=plsc.VectorSubcoreMesh(core_axis_name="c", subcore_axis_name="s"),
    scratch_types=[
        pltpu.VMEM((2, _SCH), jnp.int32),
        pltpu.VMEM((_CAP,), jnp.int32),
        pltpu.VMEM((_CAP,), jnp.int32),
        pltpu.VMEM((_CAP,), jnp.int32),
        pltpu.VMEM((_CAP,), jnp.int32),
        pltpu.VMEM((16,), jnp.int32),
    ],
    compiler_params=_SC_PARAMS_NL,
)


def _edge_body(kd_hbm, qv_hbm, srcp_hbm, dstp_hbm, cnts_hbm, out_hbm,
               src_v, dst_v, idx_v, cnt_b, krows, qvrows, acc, agg_sh,
               semg0):
    c = lax.axis_index("c")
    s = lax.axis_index("s")
    base_c = c * _HALF

    _zero_vmem(acc, _CH, _H)
    def zc(k, _):
        off = pl.multiple_of(s * _STRIPE + k * _CH, 8)
        pltpu.sync_copy(acc, agg_sh.at[pl.ds(off, _CH)])
        return 0
    lax.fori_loop(0, _STRIPE // _CH, zc, 0)
    zo = pl.multiple_of(s * _STRIPE + (_STRIPE // _CH) * _CH, 8)
    pltpu.sync_copy(acc.at[pl.ds(0, _STRIPE - (_STRIPE // _CH) * _CH)],
                    agg_sh.at[pl.ds(zo, _STRIPE - (_STRIPE // _CH) * _CH)])
    plsc.subcore_barrier()

    lane = lax.broadcasted_iota(jnp.int32, (16,), 0)
    for sti in range(2):
        st = 2 * s + sti
        pltpu.sync_copy(cnts_hbm.at[st], cnt_b)
        nch = jnp.sum(jnp.where(lane == c, cnt_b[pl.ds(0, 16)], 0))
        def chunk(ci, _):
            off = pl.ds(pl.multiple_of(ci * _CH, 8), _CH)
            pltpu.sync_copy(srcp_hbm.at[c, st, off], src_v)
            pltpu.sync_copy(dstp_hbm.at[c, st, off], dst_v)
            cp1 = pltpu.async_copy(kd_hbm.at[dst_v], krows, semg0)
            cp2 = pltpu.async_copy(qv_hbm.at[src_v], qvrows, semg0)
            def ix(r, _):
                o = pl.multiple_of(r * 16, 16)
                d = dst_v[pl.ds(o, 16)]
                ld = d - base_c
                m = (ld >= 0) & (ld < _HALF)
                idx_v[pl.ds(o, 16)] = jnp.where(m, ld, _HALF)
                return 0
            lax.fori_loop(0, _CH // 16, ix, 0)
            cp1.wait()
            cp2.wait()
            @plsc.parallel_loop(0, _CH, unroll=4)
            def _(e):
                for j in range(_H // 16):
                    kj = krows[e, pl.ds(16 * j, 16)]
                    qj = qvrows[e, pl.ds(16 * j, 16)]
                    vj = qvrows[e, pl.ds(_H + 16 * j, 16)]
                    g = 1.0 / (1.0 + jnp.exp(-(kj + qj)))
                    acc[e, pl.ds(16 * j, 16)] = g * vj
            pltpu.sync_copy(acc, agg_sh.at[idx_v], add=True)
            return 0
        lax.fori_loop(0, nch, chunk, 0)
    plsc.subcore_barrier()

    tail = _HALF - (_NSUB - 1) * _STRIPE
    so = pl.multiple_of(s * _STRIPE, 8)
    pltpu.sync_copy(agg_sh.at[pl.ds(so, tail)],
                    out_hbm.at[pl.ds(base_c + so, tail)])
    @pl.when(s < _NSUB - 1)
    def _():
        o2 = pl.multiple_of(s * _STRIPE + tail, 8)
        pltpu.sync_copy(agg_sh.at[pl.ds(o2, _STRIPE - tail)],
                        out_hbm.at[pl.ds(base_c + o2, _STRIPE - tail)])


_edge_call = functools.partial(
    pl.kernel,
    _edge_body,
    out_type=jax.ShapeDtypeStruct((_N, _H), jnp.float32),
    mesh=plsc.VectorSubcoreMesh(core_axis_name="c", subcore_axis_name="s"),
    scratch_types=[
        pltpu.VMEM((_CH,), jnp.int32),
        pltpu.VMEM((_CH,), jnp.int32),
        pltpu.VMEM((_CH,), jnp.int32),
        pltpu.VMEM((16,), jnp.int32),
        pltpu.VMEM((_CH, _H), jnp.float32),
        pltpu.VMEM((_CH, 2 * _H), jnp.float32),
        pltpu.VMEM((_CH, _H), jnp.float32),
        pltpu.VMEM_SHARED((_AGG_ROWS, _H), jnp.float32),
        pltpu.SemaphoreType.DMA,
    ],
    compiler_params=_SC_PARAMS_NL,
)


def _pool_body(agg_hbm, sk_hbm, batch_hbm, sums_hbm, cnts_hbm,
               b_v, rows_a, rows_s, ones_v, zbuf2, sums_sh, cnts_sh):
    c = lax.axis_index("c")
    s = lax.axis_index("s")
    wid = c * _NSUB + s

    _zero_vmem(rows_a, 32, _H)
    _zero_vmem(zbuf2, 32, 16)
    def fill_ones(r, _):
        ones_v[r, pl.ds(0, 16)] = jnp.full((16,), 1.0, jnp.float32)
        return 0
    lax.fori_loop(0, _PCH, fill_ones, 0)
    so = pl.multiple_of(s * 32, 8)
    pltpu.sync_copy(rows_a.at[pl.ds(0, 32)], sums_sh.at[pl.ds(so, 32)])
    pltpu.sync_copy(zbuf2, cnts_sh.at[pl.ds(so, 32)])
    plsc.subcore_barrier()

    def step(k, _):
        ch = k * (_NCORES * _NSUB) + wid
        @pl.when(ch < _PNCH)
        def _():
            off = pl.multiple_of(ch * _PCH, 8)
            pltpu.sync_copy(agg_hbm.at[pl.ds(off, _PCH)], rows_a)
            pltpu.sync_copy(sk_hbm.at[pl.ds(off, _PCH)], rows_s)
            pltpu.sync_copy(batch_hbm.at[pl.ds(off, _PCH)], b_v)
            def addrow(e, _):
                for j in range(_H // 16):
                    rows_a[e, pl.ds(16 * j, 16)] = (
                        rows_a[e, pl.ds(16 * j, 16)] + rows_s[e, pl.ds(16 * j, 16)])
                return 0
            lax.fori_loop(0, _PCH, addrow, 0)
            pltpu.sync_copy(rows_a, sums_sh.at[b_v], add=True)
            pltpu.sync_copy(ones_v, cnts_sh.at[b_v], add=True)
        return 0
    lax.fori_loop(0, _PK, step, 0)
    plsc.subcore_barrier()

    @pl.when(s == 0)
    def _():
        pltpu.sync_copy(sums_sh, sums_hbm.at[c])
        pltpu.sync_copy(cnts_sh, cnts_hbm.at[c])


_pool_call = functools.partial(
    pl.kernel,
    _pool_body,
    out_type=(jax.ShapeDtypeStruct((_NCORES, _G, _H), jnp.float32),
              jax.ShapeDtypeStruct((_NCORES, _G, 16), jnp.float32)),
    mesh=plsc.VectorSubcoreMesh(core_axis_name="c", subcore_axis_name="s"),
    scratch_types=[
        pltpu.VMEM((_PCH,), jnp.int32),
        pltpu.VMEM((_PCH, _H), jnp.float32),
        pltpu.VMEM((_PCH, _H), jnp.float32),
        pltpu.VMEM((_PCH, 16), jnp.float32),
        pltpu.VMEM((32, 16), jnp.float32),
        pltpu.VMEM_SHARED((_G, _H), jnp.float32),
        pltpu.VMEM_SHARED((_G, 16), jnp.float32),
    ],
    compiler_params=_SC_PARAMS,
)


_RB = 400
_NB = _N // _RB


def _dense1_body(x_ref, wk, bk, wqv, bqv, ws, bs, kd, qv, sk):
    X = x_ref[...]
    kd[...] = X * wk[...] + bk[...]
    qv[...] = X * wqv[...] + bqv[...]
    sk[...] = X * ws[...] + bs[...]


def _dense2_body(agg_ref, skin_ref, wk, bk, wqv, bqv, ws, bs, kd, qv, sk):
    X = jnp.maximum(agg_ref[...] + skin_ref[...], 0.0)
    kd[...] = jnp.dot(X, wk[...], preferred_element_type=jnp.float32) + bk[...]
    qv[...] = jnp.dot(X, wqv[...], preferred_element_type=jnp.float32) + bqv[...]
    sk[...] = jnp.dot(X, ws[...], preferred_element_type=jnp.float32) + bs[...]


def _wspec(din, dout):
    return pl.BlockSpec((din, dout), lambda i: (0, 0))


def _nspec(width):
    return pl.BlockSpec((_RB, width), lambda i: (i, 0))


def _dense_out_types():
    return (jax.ShapeDtypeStruct((_N, _H), jnp.float32),
            jax.ShapeDtypeStruct((_N, 2 * _H), jnp.float32),
            jax.ShapeDtypeStruct((_N, _H), jnp.float32))


def _dense1(x, wk, bk, wqv, bqv, ws, bs):
    return pl.pallas_call(
        _dense1_body,
        grid=(_NB,),
        in_specs=[_nspec(1), _wspec(1, _H), _wspec(1, _H), _wspec(1, 2 * _H),
                  _wspec(1, 2 * _H), _wspec(1, _H), _wspec(1, _H)],
        out_specs=[_nspec(_H), _nspec(2 * _H), _nspec(_H)],
        out_shape=_dense_out_types(),
    )(x, wk, bk, wqv, bqv, ws, bs)


def _dense2(agg, skin, wk, bk, wqv, bqv, ws, bs):
    return pl.pallas_call(
        _dense2_body,
        grid=(_NB,),
        in_specs=[_nspec(_H), _nspec(_H), _wspec(_H, _H), _wspec(1, _H),
                  _wspec(_H, 2 * _H), _wspec(1, 2 * _H), _wspec(_H, _H),
                  _wspec(1, _H)],
        out_specs=[_nspec(_H), _nspec(2 * _H), _nspec(_H)],
        out_shape=_dense_out_types(),
    )(agg, skin, wk, bk, wqv, bqv, ws, bs)


def _final_body(sums_ref, cnts_ref, w, b, out_ref):
    sm = sums_ref[0] + sums_ref[1]
    cn = cnts_ref[0] + cnts_ref[1]
    cnt = cn[:, 0:1]
    pooled = sm / jnp.maximum(cnt, 1.0)
    out_ref[...] = jnp.dot(pooled, w[...], preferred_element_type=jnp.float32) + b[...]


def _final(sums, cnts, w, b):
    return pl.pallas_call(
        _final_body,
        grid=(1,),
        in_specs=[pl.BlockSpec((_NCORES, _G, _H), lambda i: (0, 0, 0)),
                  pl.BlockSpec((_NCORES, _G, 16), lambda i: (0, 0, 0)),
                  _wspec(_H, _C), _wspec(1, _C)],
        out_specs=pl.BlockSpec((_G, _C), lambda i: (0, 0)),
        out_shape=jax.ShapeDtypeStruct((_G, _C), jnp.float32),
    )(sums, cnts, w, b)


def _conv_weights(p):
    wk = p["key"]["W"]
    bk = p["key"]["b"].reshape(1, -1)
    wqv = jnp.concatenate([p["query"]["W"], p["value"]["W"]], axis=1)
    bqv = jnp.concatenate([p["query"]["b"], p["value"]["b"]]).reshape(1, -1)
    ws = p["skip"]["W"]
    bs = p["skip"]["b"].reshape(1, -1)
    return wk, bk, wqv, bqv, ws, bs


def kernel(x, edge_index, batch, params):
    pad_e = jnp.concatenate(
        [edge_index,
         jnp.concatenate([jnp.zeros((1, _E2 - _E), jnp.int32),
                          jnp.full((1, _E2 - _E), _N, jnp.int32)])], axis=1)
    e3 = pad_e.reshape(2, _E2 // _SCH, _SCH).transpose(1, 0, 2)
    srcp, dstp, cnts = _part_call()(e3)

    kd, qv, sk = _dense1(x, *_conv_weights(params["conv1"]))
    agg = _edge_call()(kd, qv, srcp, dstp, cnts)
    for name in ("conv2", "conv2_1", "conv3"):
        kd, qv, sk = _dense2(agg, sk, *_conv_weights(params[name]))
        agg = _edge_call()(kd, qv, srcp, dstp, cnts)

    sums, cnts = _pool_call()(agg, sk, batch)
    return _final(sums, cnts, params["lin"]["W"], params["lin"]["b"].reshape(1, -1))


# speedup vs baseline: 5.8469x; 1.1385x over previous
import functools

import jax
import jax.numpy as jnp
from jax import lax
from jax.experimental import pallas as pl
from jax.experimental.pallas import tpu as pltpu
import jax.experimental.pallas.tpu_sc as plsc

_N = 50000
_E = 800000
_H = 64
_G = 512
_C = 10

_NCORES = 2
_NSUB = 16
_HALF = _N // _NCORES
_STRIPE = 1568
_AGG_ROWS = _NSUB * _STRIPE
_CH = 48
_E2 = 800256
_SCH = 48
_SSEG = _E2 // (_NCORES * _NSUB * _SCH)
_CAP = 25056

_PCH = 80
_PNCH = _N // _PCH
_PK = -(-_PNCH // (_NCORES * _NSUB))

_SC_PARAMS = pltpu.CompilerParams(use_tc_tiling_on_sc=False)
_SC_PARAMS_NL = pltpu.CompilerParams(use_tc_tiling_on_sc=False,
                                     needs_layout_passes=False)


def _zero_vmem(ref, rows, width):
    def body(r, _):
        for j in range(width // 16):
            ref[r, pl.ds(16 * j, 16)] = jnp.zeros((16,), jnp.float32)
        return 0
    lax.fori_loop(0, rows, body, 0)


def _part_body(e3_hbm, srcp_hbm, dstp_hbm, cnts_hbm,
               chunk_v, b0s, b0d, b1s, b1d, cntv):
    c = lax.axis_index("c")
    s = lax.axis_index("s")
    t = c * _NSUB + s
    lane16 = lax.broadcasted_iota(jnp.int32, (16,), 0)

    def scan_chunk(ci, carry):
        pltpu.sync_copy(e3_hbm.at[t * _SSEG + ci], chunk_v)
        c0, c1 = carry
        for g in range(_SCH // 16):
            sv = chunk_v[0, pl.ds(16 * g, 16)]
            dv = chunk_v[1, pl.ds(16 * g, 16)]
            m0 = dv < _HALF
            mi = jnp.where(m0, 1, 0)
            pos0 = c0 + plsc.cumsum(mi) - 1
            pos1 = c1 + plsc.cumsum(1 - mi) - 1
            plsc.store_scatter(b0s, [pos0], sv, mask=m0)
            plsc.store_scatter(b0d, [pos0], dv, mask=m0)
            m1 = jnp.logical_not(m0)
            plsc.store_scatter(b1s, [pos1], sv, mask=m1)
            plsc.store_scatter(b1d, [pos1], dv, mask=m1)
            n0 = jnp.sum(mi)
            c0 = c0 + n0
            c1 = c1 + (16 - n0)
        return (c0, c1)
    cnt0, cnt1 = lax.fori_loop(0, _SSEG, scan_chunk,
                               (jnp.int32(0), jnp.int32(0)))

    zero16 = jnp.zeros((16,), jnp.int32)
    garb16 = jnp.full((16,), _N, jnp.int32)
    for k in range(_CH // 16):
        plsc.store_scatter(b0s, [cnt0 + lane16 + 16 * k], zero16)
        plsc.store_scatter(b0d, [cnt0 + lane16 + 16 * k], garb16)
        plsc.store_scatter(b1s, [cnt1 + lane16 + 16 * k], zero16)
        plsc.store_scatter(b1d, [cnt1 + lane16 + 16 * k], garb16)
    c0p = ((cnt0 + _CH - 1) // _CH) * _CH
    c1p = ((cnt1 + _CH - 1) // _CH) * _CH

    pltpu.sync_copy(b0s, srcp_hbm.at[0, t])
    pltpu.sync_copy(b0d, dstp_hbm.at[0, t])
    pltpu.sync_copy(b1s, srcp_hbm.at[1, t])
    pltpu.sync_copy(b1d, dstp_hbm.at[1, t])
    cntv[pl.ds(0, 16)] = jnp.where(lane16 == 0, c0p // _CH,
                                   jnp.where(lane16 == 1, c1p // _CH, 0))
    pltpu.sync_copy(cntv, cnts_hbm.at[t])


_part_call = functools.partial(
    pl.kernel,
    _part_body,
    out_type=(jax.ShapeDtypeStruct((_NCORES, _NCORES * _NSUB, _CAP), jnp.int32),
              jax.ShapeDtypeStruct((_NCORES, _NCORES * _NSUB, _CAP), jnp.int32),
              jax.ShapeDtypeStruct((_NCORES * _NSUB, 16), jnp.int32)),
    mesh=plsc.VectorSubcoreMesh(core_axis_name="c", subcore_axis_name="s"),
    scratch_types=[
        pltpu.VMEM((2, _SCH), jnp.int32),
        pltpu.VMEM((_CAP,), jnp.int32),
        pltpu.VMEM((_CAP,), jnp.int32),
        pltpu.VMEM((_CAP,), jnp.int32),
        pltpu.VMEM((_CAP,), jnp.int32),
        pltpu.VMEM((16,), jnp.int32),
    ],
    compiler_params=_SC_PARAMS_NL,
)


def _edge_body(kd_hbm, qv_hbm, srcp_hbm, dstp_hbm, cnts_hbm, out_hbm,
               srcA, dstA, srcB, dstB, idxA, idxB, cnt_b,
               krowsA, qvrowsA, krowsB, qvrowsB, acc, agg_sh,
               semA, semB):
    c = lax.axis_index("c")
    s = lax.axis_index("s")
    base_c = c * _HALF

    _zero_vmem(acc, _CH, _H)
    def zc(k, _):
        off = pl.multiple_of(s * _STRIPE + k * _CH, 8)
        pltpu.sync_copy(acc, agg_sh.at[pl.ds(off, _CH)])
        return 0
    lax.fori_loop(0, _STRIPE // _CH, zc, 0)
    zo = pl.multiple_of(s * _STRIPE + (_STRIPE // _CH) * _CH, 8)
    pltpu.sync_copy(acc.at[pl.ds(0, _STRIPE - (_STRIPE // _CH) * _CH)],
                    agg_sh.at[pl.ds(zo, _STRIPE - (_STRIPE // _CH) * _CH)])
    plsc.subcore_barrier()

    def compute_ix(dv, iv):
        def ix(r, _):
            o = pl.multiple_of(r * 16, 16)
            d = dv[pl.ds(o, 16)]
            ld = d - base_c
            m = (ld >= 0) & (ld < _HALF)
            iv[pl.ds(o, 16)] = jnp.where(m, ld, _HALF)
            return 0
        lax.fori_loop(0, _CH // 16, ix, 0)

    def gate(krows, qvrows):
        @plsc.parallel_loop(0, _CH, unroll=4)
        def _(e):
            for j in range(_H // 16):
                kj = krows[e, pl.ds(16 * j, 16)]
                qj = qvrows[e, pl.ds(16 * j, 16)]
                vj = qvrows[e, pl.ds(_H + 16 * j, 16)]
                g = 1.0 / (1.0 + jnp.exp(-(kj + qj)))
                acc[e, pl.ds(16 * j, 16)] = g * vj

    lane = lax.broadcasted_iota(jnp.int32, (16,), 0)
    for sti in range(2):
        st = 2 * s + sti
        pltpu.sync_copy(cnts_hbm.at[st], cnt_b)
        nch = jnp.sum(jnp.where(lane == c, cnt_b[pl.ds(0, 16)], 0))

        def load_fire(ci, sv, dv, kr, qr, sem):
            off = pl.ds(pl.multiple_of(ci * _CH, 8), _CH)
            pltpu.sync_copy(srcp_hbm.at[c, st, off], sv)
            pltpu.sync_copy(dstp_hbm.at[c, st, off], dv)
            return (pltpu.async_copy(kd_hbm.at[dv], kr, sem),
                    pltpu.async_copy(qv_hbm.at[sv], qr, sem))

        def pairf(p, _):
            ca1, ca2 = load_fire(2 * p, srcA, dstA, krowsA, qvrowsA, semA)
            cb1, cb2 = load_fire(2 * p + 1, srcB, dstB, krowsB, qvrowsB, semB)
            ca1.wait()
            ca2.wait()
            compute_ix(dstA, idxA)
            gate(krowsA, qvrowsA)
            cb1.wait()
            cb2.wait()
            pltpu.sync_copy(acc, agg_sh.at[idxA], add=True)
            compute_ix(dstB, idxB)
            gate(krowsB, qvrowsB)
            pltpu.sync_copy(acc, agg_sh.at[idxB], add=True)
            return 0
        lax.fori_loop(0, nch // 2, pairf, 0)

        @pl.when(lax.rem(nch, 2) == 1)
        def _():
            ca1, ca2 = load_fire(nch - 1, srcA, dstA, krowsA, qvrowsA, semA)
            ca1.wait()
            ca2.wait()
            compute_ix(dstA, idxA)
            gate(krowsA, qvrowsA)
            pltpu.sync_copy(acc, agg_sh.at[idxA], add=True)
    plsc.subcore_barrier()

    tail = _HALF - (_NSUB - 1) * _STRIPE
    so = pl.multiple_of(s * _STRIPE, 8)
    pltpu.sync_copy(agg_sh.at[pl.ds(so, tail)],
                    out_hbm.at[pl.ds(base_c + so, tail)])
    @pl.when(s < _NSUB - 1)
    def _():
        o2 = pl.multiple_of(s * _STRIPE + tail, 8)
        pltpu.sync_copy(agg_sh.at[pl.ds(o2, _STRIPE - tail)],
                        out_hbm.at[pl.ds(base_c + o2, _STRIPE - tail)])


_edge_call = functools.partial(
    pl.kernel,
    _edge_body,
    out_type=jax.ShapeDtypeStruct((_N, _H), jnp.float32),
    mesh=plsc.VectorSubcoreMesh(core_axis_name="c", subcore_axis_name="s"),
    scratch_types=[
        pltpu.VMEM((_CH,), jnp.int32),
        pltpu.VMEM((_CH,), jnp.int32),
        pltpu.VMEM((_CH,), jnp.int32),
        pltpu.VMEM((_CH,), jnp.int32),
        pltpu.VMEM((_CH,), jnp.int32),
        pltpu.VMEM((_CH,), jnp.int32),
        pltpu.VMEM((16,), jnp.int32),
        pltpu.VMEM((_CH, _H), jnp.float32),
        pltpu.VMEM((_CH, 2 * _H), jnp.float32),
        pltpu.VMEM((_CH, _H), jnp.float32),
        pltpu.VMEM((_CH, 2 * _H), jnp.float32),
        pltpu.VMEM((_CH, _H), jnp.float32),
        pltpu.VMEM_SHARED((_AGG_ROWS, _H), jnp.float32),
        pltpu.SemaphoreType.DMA,
        pltpu.SemaphoreType.DMA,
    ],
    compiler_params=_SC_PARAMS_NL,
)


def _pool_body(agg_hbm, sk_hbm, batch_hbm, sums_hbm, cnts_hbm,
               b_v, rows_a, rows_s, ones_v, zbuf2, sums_sh, cnts_sh):
    c = lax.axis_index("c")
    s = lax.axis_index("s")
    wid = c * _NSUB + s

    _zero_vmem(rows_a, 32, _H)
    _zero_vmem(zbuf2, 32, 16)
    def fill_ones(r, _):
        ones_v[r, pl.ds(0, 16)] = jnp.full((16,), 1.0, jnp.float32)
        return 0
    lax.fori_loop(0, _PCH, fill_ones, 0)
    so = pl.multiple_of(s * 32, 8)
    pltpu.sync_copy(rows_a.at[pl.ds(0, 32)], sums_sh.at[pl.ds(so, 32)])
    pltpu.sync_copy(zbuf2, cnts_sh.at[pl.ds(so, 32)])
    plsc.subcore_barrier()

    def step(k, _):
        ch = k * (_NCORES * _NSUB) + wid
        @pl.when(ch < _PNCH)
        def _():
            off = pl.multiple_of(ch * _PCH, 8)
            pltpu.sync_copy(agg_hbm.at[pl.ds(off, _PCH)], rows_a)
            pltpu.sync_copy(sk_hbm.at[pl.ds(off, _PCH)], rows_s)
            pltpu.sync_copy(batch_hbm.at[pl.ds(off, _PCH)], b_v)
            def addrow(e, _):
                for j in range(_H // 16):
                    rows_a[e, pl.ds(16 * j, 16)] = (
                        rows_a[e, pl.ds(16 * j, 16)] + rows_s[e, pl.ds(16 * j, 16)])
                return 0
            lax.fori_loop(0, _PCH, addrow, 0)
            pltpu.sync_copy(rows_a, sums_sh.at[b_v], add=True)
            pltpu.sync_copy(ones_v, cnts_sh.at[b_v], add=True)
        return 0
    lax.fori_loop(0, _PK, step, 0)
    plsc.subcore_barrier()

    @pl.when(s == 0)
    def _():
        pltpu.sync_copy(sums_sh, sums_hbm.at[c])
        pltpu.sync_copy(cnts_sh, cnts_hbm.at[c])


_pool_call = functools.partial(
    pl.kernel,
    _pool_body,
    out_type=(jax.ShapeDtypeStruct((_NCORES, _G, _H), jnp.float32),
              jax.ShapeDtypeStruct((_NCORES, _G, 16), jnp.float32)),
    mesh=plsc.VectorSubcoreMesh(core_axis_name="c", subcore_axis_name="s"),
    scratch_types=[
        pltpu.VMEM((_PCH,), jnp.int32),
        pltpu.VMEM((_PCH, _H), jnp.float32),
        pltpu.VMEM((_PCH, _H), jnp.float32),
        pltpu.VMEM((_PCH, 16), jnp.float32),
        pltpu.VMEM((32, 16), jnp.float32),
        pltpu.VMEM_SHARED((_G, _H), jnp.float32),
        pltpu.VMEM_SHARED((_G, 16), jnp.float32),
    ],
    compiler_params=_SC_PARAMS,
)


_RB = 400
_NB = _N // _RB


def _dense1_body(x_ref, wk, bk, wqv, bqv, ws, bs, kd, qv, sk):
    X = x_ref[...]
    kd[...] = X * wk[...] + bk[...]
    qv[...] = X * wqv[...] + bqv[...]
    sk[...] = X * ws[...] + bs[...]


def _dense2_body(agg_ref, skin_ref, wk, bk, wqv, bqv, ws, bs, kd, qv, sk):
    X = jnp.maximum(agg_ref[...] + skin_ref[...], 0.0)
    kd[...] = jnp.dot(X, wk[...], preferred_element_type=jnp.float32) + bk[...]
    qv[...] = jnp.dot(X, wqv[...], preferred_element_type=jnp.float32) + bqv[...]
    sk[...] = jnp.dot(X, ws[...], preferred_element_type=jnp.float32) + bs[...]


def _wspec(din, dout):
    return pl.BlockSpec((din, dout), lambda i: (0, 0))


def _nspec(width):
    return pl.BlockSpec((_RB, width), lambda i: (i, 0))


def _dense_out_types():
    return (jax.ShapeDtypeStruct((_N, _H), jnp.float32),
            jax.ShapeDtypeStruct((_N, 2 * _H), jnp.float32),
            jax.ShapeDtypeStruct((_N, _H), jnp.float32))


def _dense1(x, wk, bk, wqv, bqv, ws, bs):
    return pl.pallas_call(
        _dense1_body,
        grid=(_NB,),
        in_specs=[_nspec(1), _wspec(1, _H), _wspec(1, _H), _wspec(1, 2 * _H),
                  _wspec(1, 2 * _H), _wspec(1, _H), _wspec(1, _H)],
        out_specs=[_nspec(_H), _nspec(2 * _H), _nspec(_H)],
        out_shape=_dense_out_types(),
    )(x, wk, bk, wqv, bqv, ws, bs)


def _dense2(agg, skin, wk, bk, wqv, bqv, ws, bs):
    return pl.pallas_call(
        _dense2_body,
        grid=(_NB,),
        in_specs=[_nspec(_H), _nspec(_H), _wspec(_H, _H), _wspec(1, _H),
                  _wspec(_H, 2 * _H), _wspec(1, 2 * _H), _wspec(_H, _H),
                  _wspec(1, _H)],
        out_specs=[_nspec(_H), _nspec(2 * _H), _nspec(_H)],
        out_shape=_dense_out_types(),
    )(agg, skin, wk, bk, wqv, bqv, ws, bs)


def _final_body(sums_ref, cnts_ref, w, b, out_ref):
    sm = sums_ref[0] + sums_ref[1]
    cn = cnts_ref[0] + cnts_ref[1]
    cnt = cn[:, 0:1]
    pooled = sm / jnp.maximum(cnt, 1.0)
    out_ref[...] = jnp.dot(pooled, w[...], preferred_element_type=jnp.float32) + b[...]


def _final(sums, cnts, w, b):
    return pl.pallas_call(
        _final_body,
        grid=(1,),
        in_specs=[pl.BlockSpec((_NCORES, _G, _H), lambda i: (0, 0, 0)),
                  pl.BlockSpec((_NCORES, _G, 16), lambda i: (0, 0, 0)),
                  _wspec(_H, _C), _wspec(1, _C)],
        out_specs=pl.BlockSpec((_G, _C), lambda i: (0, 0)),
        out_shape=jax.ShapeDtypeStruct((_G, _C), jnp.float32),
    )(sums, cnts, w, b)


def _conv_weights(p):
    wk = p["key"]["W"]
    bk = p["key"]["b"].reshape(1, -1)
    wqv = jnp.concatenate([p["query"]["W"], p["value"]["W"]], axis=1)
    bqv = jnp.concatenate([p["query"]["b"], p["value"]["b"]]).reshape(1, -1)
    ws = p["skip"]["W"]
    bs = p["skip"]["b"].reshape(1, -1)
    return wk, bk, wqv, bqv, ws, bs


def kernel(x, edge_index, batch, params):
    pad_e = jnp.concatenate(
        [edge_index,
         jnp.concatenate([jnp.zeros((1, _E2 - _E), jnp.int32),
                          jnp.full((1, _E2 - _E), _N, jnp.int32)])], axis=1)
    e3 = pad_e.reshape(2, _E2 // _SCH, _SCH).transpose(1, 0, 2)
    srcp, dstp, cnts = _part_call()(e3)

    kd, qv, sk = _dense1(x, *_conv_weights(params["conv1"]))
    agg = _edge_call()(kd, qv, srcp, dstp, cnts)
    for name in ("conv2", "conv2_1", "conv3"):
        kd, qv, sk = _dense2(agg, sk, *_conv_weights(params[name]))
        agg = _edge_call()(kd, qv, srcp, dstp, cnts)

    sums, cnts = _pool_call()(agg, sk, batch)
    return _final(sums, cnts, params["lin"]["W"], params["lin"]["b"].reshape(1, -1))


# speedup vs baseline: 6.9497x; 1.1886x over previous
import functools

import jax
import jax.numpy as jnp
from jax import lax
from jax.experimental import pallas as pl
from jax.experimental.pallas import tpu as pltpu
import jax.experimental.pallas.tpu_sc as plsc

_N = 50000
_E = 800000
_H = 64
_G = 512
_C = 10

_NCORES = 2
_NSUB = 16
_HALF = _N // _NCORES
_STRIPE = 1568
_AGG_ROWS = _NSUB * _STRIPE
_CH = 48
_E2 = 800256
_SCH = 48
_SSEG = _E2 // (_NCORES * _NSUB * _SCH)
_CAP = 25056

_PCH = 80
_PNCH = _N // _PCH
_PK = -(-_PNCH // (_NCORES * _NSUB))

_SC_PARAMS = pltpu.CompilerParams(use_tc_tiling_on_sc=False)
_SC_PARAMS_NL = pltpu.CompilerParams(use_tc_tiling_on_sc=False,
                                     needs_layout_passes=False)


def _zero_vmem(ref, rows, width):
    def body(r, _):
        for j in range(width // 16):
            ref[r, pl.ds(16 * j, 16)] = jnp.zeros((16,), jnp.float32)
        return 0
    lax.fori_loop(0, rows, body, 0)


def _part_body(e3_hbm, sd_hbm, cnts_hbm,
               chunk_v, b0s, b0d, b1s, b1d, cntv):
    c = lax.axis_index("c")
    s = lax.axis_index("s")
    t = c * _NSUB + s
    lane16 = lax.broadcasted_iota(jnp.int32, (16,), 0)

    def scan_chunk(ci, carry):
        pltpu.sync_copy(e3_hbm.at[t * _SSEG + ci], chunk_v)
        c0, c1 = carry
        for g in range(_SCH // 16):
            sv = chunk_v[0, pl.ds(16 * g, 16)]
            dv = chunk_v[1, pl.ds(16 * g, 16)]
            m0 = dv < _HALF
            mi = jnp.where(m0, 1, 0)
            pos0 = c0 + plsc.cumsum(mi) - 1
            pos1 = c1 + plsc.cumsum(1 - mi) - 1
            plsc.store_scatter(b0s, [pos0], sv, mask=m0)
            plsc.store_scatter(b0d, [pos0], dv, mask=m0)
            m1 = jnp.logical_not(m0)
            plsc.store_scatter(b1s, [pos1], sv, mask=m1)
            plsc.store_scatter(b1d, [pos1], dv, mask=m1)
            n0 = jnp.sum(mi)
            c0 = c0 + n0
            c1 = c1 + (16 - n0)
        return (c0, c1)
    cnt0, cnt1 = lax.fori_loop(0, _SSEG, scan_chunk,
                               (jnp.int32(0), jnp.int32(0)))

    zero16 = jnp.zeros((16,), jnp.int32)
    garb16 = jnp.full((16,), _N, jnp.int32)
    for k in range(_CH // 16):
        plsc.store_scatter(b0s, [cnt0 + lane16 + 16 * k], zero16)
        plsc.store_scatter(b0d, [cnt0 + lane16 + 16 * k], garb16)
        plsc.store_scatter(b1s, [cnt1 + lane16 + 16 * k], zero16)
        plsc.store_scatter(b1d, [cnt1 + lane16 + 16 * k], garb16)
    c0p = ((cnt0 + _CH - 1) // _CH) * _CH
    c1p = ((cnt1 + _CH - 1) // _CH) * _CH

    pltpu.sync_copy(b0s, sd_hbm.at[0, t, 0])
    pltpu.sync_copy(b0d, sd_hbm.at[0, t, 1])
    pltpu.sync_copy(b1s, sd_hbm.at[1, t, 0])
    pltpu.sync_copy(b1d, sd_hbm.at[1, t, 1])
    cntv[pl.ds(0, 16)] = jnp.where(lane16 == 0, c0p // _CH,
                                   jnp.where(lane16 == 1, c1p // _CH, 0))
    pltpu.sync_copy(cntv, cnts_hbm.at[t])


_part_call = functools.partial(
    pl.kernel,
    _part_body,
    out_type=(jax.ShapeDtypeStruct((_NCORES, _NCORES * _NSUB, 2, _CAP),
                                    jnp.int32),
              jax.ShapeDtypeStruct((_NCORES * _NSUB, 16), jnp.int32)),
    mesh=plsc.VectorSubcoreMesh(core_axis_name="c", subcore_axis_name="s"),
    scratch_types=[
        pltpu.VMEM((2, _SCH), jnp.int32),
        pltpu.VMEM((_CAP,), jnp.int32),
        pltpu.VMEM((_CAP,), jnp.int32),
        pltpu.VMEM((_CAP,), jnp.int32),
        pltpu.VMEM((_CAP,), jnp.int32),
        pltpu.VMEM((16,), jnp.int32),
    ],
    compiler_params=_SC_PARAMS_NL,
)


def _edge_body(kd_hbm, qv_hbm, sd_hbm, cnts_hbm, out_hbm,
               sdA, sdB, idxA, idxB, cnt_b,
               krowsA, qvrowsA, krowsB, qvrowsB, acc, agg_sh,
               semA, semB):
    c = lax.axis_index("c")
    s = lax.axis_index("s")
    base_c = c * _HALF

    _zero_vmem(acc, _CH, _H)
    def zc(k, _):
        off = pl.multiple_of(s * _STRIPE + k * _CH, 8)
        pltpu.sync_copy(acc, agg_sh.at[pl.ds(off, _CH)])
        return 0
    lax.fori_loop(0, _STRIPE // _CH, zc, 0)
    zo = pl.multiple_of(s * _STRIPE + (_STRIPE // _CH) * _CH, 8)
    pltpu.sync_copy(acc.at[pl.ds(0, _STRIPE - (_STRIPE // _CH) * _CH)],
                    agg_sh.at[pl.ds(zo, _STRIPE - (_STRIPE // _CH) * _CH)])
    plsc.subcore_barrier()

    def compute_ix(sd, iv):
        def ix(r, _):
            o = pl.multiple_of(r * 16, 16)
            d = sd[1, pl.ds(o, 16)]
            ld = d - base_c
            m = (ld >= 0) & (ld < _HALF)
            iv[pl.ds(o, 16)] = jnp.where(m, ld, _HALF)
            return 0
        lax.fori_loop(0, _CH // 16, ix, 0)

    def gate(krows, qvrows):
        @plsc.parallel_loop(0, _CH, unroll=4)
        def _(e):
            for j in range(_H // 16):
                kj = krows[e, pl.ds(16 * j, 16)]
                qj = qvrows[e, pl.ds(16 * j, 16)]
                vj = qvrows[e, pl.ds(_H + 16 * j, 16)]
                g = 1.0 / (1.0 + jnp.exp(-(kj + qj)))
                acc[e, pl.ds(16 * j, 16)] = g * vj

    lane = lax.broadcasted_iota(jnp.int32, (16,), 0)
    for sti in range(2):
        st = 2 * s + sti
        pltpu.sync_copy(cnts_hbm.at[st], cnt_b)
        nch = jnp.sum(jnp.where(lane == c, cnt_b[pl.ds(0, 16)], 0))

        def load_fire(ci, sd, kr, qr, sem):
            off = pl.ds(pl.multiple_of(ci * _CH, 8), _CH)
            pltpu.sync_copy(sd_hbm.at[c, st, pl.ds(0, 2), off], sd)
            return (pltpu.async_copy(kd_hbm.at[sd.at[1]], kr, sem),
                    pltpu.async_copy(qv_hbm.at[sd.at[0]], qr, sem))

        def pairf(p, _):
            ca1, ca2 = load_fire(2 * p, sdA, krowsA, qvrowsA, semA)
            cb1, cb2 = load_fire(2 * p + 1, sdB, krowsB, qvrowsB, semB)
            ca1.wait()
            ca2.wait()
            compute_ix(sdA, idxA)
            gate(krowsA, qvrowsA)
            cb1.wait()
            cb2.wait()
            pltpu.sync_copy(acc, agg_sh.at[idxA], add=True)
            compute_ix(sdB, idxB)
            gate(krowsB, qvrowsB)
            pltpu.sync_copy(acc, agg_sh.at[idxB], add=True)
            return 0
        lax.fori_loop(0, nch // 2, pairf, 0)

        @pl.when(lax.rem(nch, 2) == 1)
        def _():
            ca1, ca2 = load_fire(nch - 1, sdA, krowsA, qvrowsA, semA)
            ca1.wait()
            ca2.wait()
            compute_ix(sdA, idxA)
            gate(krowsA, qvrowsA)
            pltpu.sync_copy(acc, agg_sh.at[idxA], add=True)
    plsc.subcore_barrier()

    tail = _HALF - (_NSUB - 1) * _STRIPE
    so = pl.multiple_of(s * _STRIPE, 8)
    pltpu.sync_copy(agg_sh.at[pl.ds(so, tail)],
                    out_hbm.at[pl.ds(base_c + so, tail)])
    @pl.when(s < _NSUB - 1)
    def _():
        o2 = pl.multiple_of(s * _STRIPE + tail, 8)
        pltpu.sync_copy(agg_sh.at[pl.ds(o2, _STRIPE - tail)],
                        out_hbm.at[pl.ds(base_c + o2, _STRIPE - tail)])


_edge_call = functools.partial(
    pl.kernel,
    _edge_body,
    out_type=jax.ShapeDtypeStruct((_N, _H), jnp.float32),
    mesh=plsc.VectorSubcoreMesh(core_axis_name="c", subcore_axis_name="s"),
    scratch_types=[
        pltpu.VMEM((2, _CH), jnp.int32),
        pltpu.VMEM((2, _CH), jnp.int32),
        pltpu.VMEM((_CH,), jnp.int32),
        pltpu.VMEM((_CH,), jnp.int32),
        pltpu.VMEM((16,), jnp.int32),
        pltpu.VMEM((_CH, _H), jnp.float32),
        pltpu.VMEM((_CH, 2 * _H), jnp.float32),
        pltpu.VMEM((_CH, _H), jnp.float32),
        pltpu.VMEM((_CH, 2 * _H), jnp.float32),
        pltpu.VMEM((_CH, _H), jnp.float32),
        pltpu.VMEM_SHARED((_AGG_ROWS, _H), jnp.float32),
        pltpu.SemaphoreType.DMA,
        pltpu.SemaphoreType.DMA,
    ],
    compiler_params=_SC_PARAMS_NL,
)


def _pool_body(agg_hbm, sk_hbm, batch_hbm, sums_hbm, cnts_hbm,
               b_v, rows_a, rows_s, ones_v, zbuf2, sums_sh, cnts_sh):
    c = lax.axis_index("c")
    s = lax.axis_index("s")
    wid = c * _NSUB + s

    _zero_vmem(rows_a, 32, _H)
    _zero_vmem(zbuf2, 32, 16)
    def fill_ones(r, _):
        ones_v[r, pl.ds(0, 16)] = jnp.full((16,), 1.0, jnp.float32)
        return 0
    lax.fori_loop(0, _PCH, fill_ones, 0)
    so = pl.multiple_of(s * 32, 8)
    pltpu.sync_copy(rows_a.at[pl.ds(0, 32)], sums_sh.at[pl.ds(so, 32)])
    pltpu.sync_copy(zbuf2, cnts_sh.at[pl.ds(so, 32)])
    plsc.subcore_barrier()

    def step(k, _):
        ch = k * (_NCORES * _NSUB) + wid
        @pl.when(ch < _PNCH)
        def _():
            off = pl.multiple_of(ch * _PCH, 8)
            pltpu.sync_copy(agg_hbm.at[pl.ds(off, _PCH)], rows_a)
            pltpu.sync_copy(sk_hbm.at[pl.ds(off, _PCH)], rows_s)
            pltpu.sync_copy(batch_hbm.at[pl.ds(off, _PCH)], b_v)
            def addrow(e, _):
                for j in range(_H // 16):
                    rows_a[e, pl.ds(16 * j, 16)] = (
                        rows_a[e, pl.ds(16 * j, 16)] + rows_s[e, pl.ds(16 * j, 16)])
                return 0
            lax.fori_loop(0, _PCH, addrow, 0)
            pltpu.sync_copy(rows_a, sums_sh.at[b_v], add=True)
            pltpu.sync_copy(ones_v, cnts_sh.at[b_v], add=True)
        return 0
    lax.fori_loop(0, _PK, step, 0)
    plsc.subcore_barrier()

    @pl.when(s == 0)
    def _():
        pltpu.sync_copy(sums_sh, sums_hbm.at[c])
        pltpu.sync_copy(cnts_sh, cnts_hbm.at[c])


_pool_call = functools.partial(
    pl.kernel,
    _pool_body,
    out_type=(jax.ShapeDtypeStruct((_NCORES, _G, _H), jnp.float32),
              jax.ShapeDtypeStruct((_NCORES, _G, 16), jnp.float32)),
    mesh=plsc.VectorSubcoreMesh(core_axis_name="c", subcore_axis_name="s"),
    scratch_types=[
        pltpu.VMEM((_PCH,), jnp.int32),
        pltpu.VMEM((_PCH, _H), jnp.float32),
        pltpu.VMEM((_PCH, _H), jnp.float32),
        pltpu.VMEM((_PCH, 16), jnp.float32),
        pltpu.VMEM((32, 16), jnp.float32),
        pltpu.VMEM_SHARED((_G, _H), jnp.float32),
        pltpu.VMEM_SHARED((_G, 16), jnp.float32),
    ],
    compiler_params=_SC_PARAMS,
)


_RB = 400
_NB = _N // _RB


def _dense1_body(x_ref, wk, bk, wqv, bqv, ws, bs, kd, qv, sk):
    X = x_ref[...]
    kd[...] = X * wk[...] + bk[...]
    qv[...] = X * wqv[...] + bqv[...]
    sk[...] = X * ws[...] + bs[...]


def _dense2_body(agg_ref, skin_ref, wk, bk, wqv, bqv, ws, bs, kd, qv, sk):
    X = jnp.maximum(agg_ref[...] + skin_ref[...], 0.0)
    kd[...] = jnp.dot(X, wk[...], preferred_element_type=jnp.float32) + bk[...]
    qv[...] = jnp.dot(X, wqv[...], preferred_element_type=jnp.float32) + bqv[...]
    sk[...] = jnp.dot(X, ws[...], preferred_element_type=jnp.float32) + bs[...]


def _wspec(din, dout):
    return pl.BlockSpec((din, dout), lambda i: (0, 0))


def _nspec(width):
    return pl.BlockSpec((_RB, width), lambda i: (i, 0))


def _dense_out_types():
    return (jax.ShapeDtypeStruct((_N, _H), jnp.float32),
            jax.ShapeDtypeStruct((_N, 2 * _H), jnp.float32),
            jax.ShapeDtypeStruct((_N, _H), jnp.float32))


def _dense1(x, wk, bk, wqv, bqv, ws, bs):
    return pl.pallas_call(
        _dense1_body,
        grid=(_NB,),
        in_specs=[_nspec(1), _wspec(1, _H), _wspec(1, _H), _wspec(1, 2 * _H),
                  _wspec(1, 2 * _H), _wspec(1, _H), _wspec(1, _H)],
        out_specs=[_nspec(_H), _nspec(2 * _H), _nspec(_H)],
        out_shape=_dense_out_types(),
    )(x, wk, bk, wqv, bqv, ws, bs)


def _dense2(agg, skin, wk, bk, wqv, bqv, ws, bs):
    return pl.pallas_call(
        _dense2_body,
        grid=(_NB,),
        in_specs=[_nspec(_H), _nspec(_H), _wspec(_H, _H), _wspec(1, _H),
                  _wspec(_H, 2 * _H), _wspec(1, 2 * _H), _wspec(_H, _H),
                  _wspec(1, _H)],
        out_specs=[_nspec(_H), _nspec(2 * _H), _nspec(_H)],
        out_shape=_dense_out_types(),
    )(agg, skin, wk, bk, wqv, bqv, ws, bs)


def _final_body(sums_ref, cnts_ref, w, b, out_ref):
    sm = sums_ref[0] + sums_ref[1]
    cn = cnts_ref[0] + cnts_ref[1]
    cnt = cn[:, 0:1]
    pooled = sm / jnp.maximum(cnt, 1.0)
    out_ref[...] = jnp.dot(pooled, w[...], preferred_element_type=jnp.float32) + b[...]


def _final(sums, cnts, w, b):
    return pl.pallas_call(
        _final_body,
        grid=(1,),
        in_specs=[pl.BlockSpec((_NCORES, _G, _H), lambda i: (0, 0, 0)),
                  pl.BlockSpec((_NCORES, _G, 16), lambda i: (0, 0, 0)),
                  _wspec(_H, _C), _wspec(1, _C)],
        out_specs=pl.BlockSpec((_G, _C), lambda i: (0, 0)),
        out_shape=jax.ShapeDtypeStruct((_G, _C), jnp.float32),
    )(sums, cnts, w, b)


def _conv_weights(p):
    wk = p["key"]["W"]
    bk = p["key"]["b"].reshape(1, -1)
    wqv = jnp.concatenate([p["query"]["W"], p["value"]["W"]], axis=1)
    bqv = jnp.concatenate([p["query"]["b"], p["value"]["b"]]).reshape(1, -1)
    ws = p["skip"]["W"]
    bs = p["skip"]["b"].reshape(1, -1)
    return wk, bk, wqv, bqv, ws, bs


def kernel(x, edge_index, batch, params):
    pad_e = jnp.concatenate(
        [edge_index,
         jnp.concatenate([jnp.zeros((1, _E2 - _E), jnp.int32),
                          jnp.full((1, _E2 - _E), _N, jnp.int32)])], axis=1)
    e3 = pad_e.reshape(2, _E2 // _SCH, _SCH).transpose(1, 0, 2)
    sd, cnts = _part_call()(e3)

    kd, qv, sk = _dense1(x, *_conv_weights(params["conv1"]))
    agg = _edge_call()(kd, qv, sd, cnts)
    for name in ("conv2", "conv2_1", "conv3"):
        kd, qv, sk = _dense2(agg, sk, *_conv_weights(params[name]))
        agg = _edge_call()(kd, qv, sd, cnts)

    sums, cnts = _pool_call()(agg, sk, batch)
    return _final(sums, cnts, params["lin"]["W"], params["lin"]["b"].reshape(1, -1))


# speedup vs baseline: 7.0807x; 1.0189x over previous
import functools

import jax
import jax.numpy as jnp
from jax import lax
from jax.experimental import pallas as pl
from jax.experimental.pallas import tpu as pltpu
import jax.experimental.pallas.tpu_sc as plsc

_N = 50000
_E = 800000
_H = 64
_G = 512
_C = 10

_NCORES = 2
_NSUB = 16
_HALF = _N // _NCORES
_STRIPE = 1568
_AGG_ROWS = _NSUB * _STRIPE
_CH = 48
_E2 = 800256
_SCH = 48
_SSEG = _E2 // (_NCORES * _NSUB * _SCH)
_CAP = 25056

_PCH = 80
_PNCH = _N // _PCH
_PK = -(-_PNCH // (_NCORES * _NSUB))

_UPERM = sum((([32 * b + i for i in range(0, 32, 2)] +
               [32 * b + i for i in range(1, 32, 2)]) for b in (0, 1)), [])

_SC_PARAMS = pltpu.CompilerParams(use_tc_tiling_on_sc=False)
_SC_PARAMS_NL = pltpu.CompilerParams(use_tc_tiling_on_sc=False,
                                     needs_layout_passes=False)


def _zero_vmem(ref, rows, width):
    def body(r, _):
        for j in range(width // 16):
            ref[r, pl.ds(16 * j, 16)] = jnp.zeros((16,), jnp.float32)
        return 0
    lax.fori_loop(0, rows, body, 0)


def _part_body(e3_hbm, sd_hbm, cnts_hbm,
               chunk_v, b0s, b0d, b1s, b1d, cntv):
    c = lax.axis_index("c")
    s = lax.axis_index("s")
    t = c * _NSUB + s
    lane16 = lax.broadcasted_iota(jnp.int32, (16,), 0)

    def scan_chunk(ci, carry):
        pltpu.sync_copy(e3_hbm.at[t * _SSEG + ci], chunk_v)
        c0, c1 = carry
        for g in range(_SCH // 16):
            sv = chunk_v[0, pl.ds(16 * g, 16)]
            dv = chunk_v[1, pl.ds(16 * g, 16)]
            m0 = dv < _HALF
            mi = jnp.where(m0, 1, 0)
            pos0 = c0 + plsc.cumsum(mi) - 1
            pos1 = c1 + plsc.cumsum(1 - mi) - 1
            plsc.store_scatter(b0s, [pos0], sv, mask=m0)
            plsc.store_scatter(b0d, [pos0], dv, mask=m0)
            m1 = jnp.logical_not(m0)
            plsc.store_scatter(b1s, [pos1], sv, mask=m1)
            plsc.store_scatter(b1d, [pos1], dv, mask=m1)
            n0 = jnp.sum(mi)
            c0 = c0 + n0
            c1 = c1 + (16 - n0)
        return (c0, c1)
    cnt0, cnt1 = lax.fori_loop(0, _SSEG, scan_chunk,
                               (jnp.int32(0), jnp.int32(0)))

    zero16 = jnp.zeros((16,), jnp.int32)
    garb16 = jnp.full((16,), _N, jnp.int32)
    for k in range(_CH // 16):
        plsc.store_scatter(b0s, [cnt0 + lane16 + 16 * k], zero16)
        plsc.store_scatter(b0d, [cnt0 + lane16 + 16 * k], garb16)
        plsc.store_scatter(b1s, [cnt1 + lane16 + 16 * k], zero16)
        plsc.store_scatter(b1d, [cnt1 + lane16 + 16 * k], garb16)
    c0p = ((cnt0 + _CH - 1) // _CH) * _CH
    c1p = ((cnt1 + _CH - 1) // _CH) * _CH

    pltpu.sync_copy(b0s, sd_hbm.at[0, t, 0])
    pltpu.sync_copy(b0d, sd_hbm.at[0, t, 1])
    pltpu.sync_copy(b1s, sd_hbm.at[1, t, 0])
    pltpu.sync_copy(b1d, sd_hbm.at[1, t, 1])
    cntv[pl.ds(0, 16)] = jnp.where(lane16 == 0, c0p // _CH,
                                   jnp.where(lane16 == 1, c1p // _CH, 0))
    pltpu.sync_copy(cntv, cnts_hbm.at[t])


_part_call = functools.partial(
    pl.kernel,
    _part_body,
    out_type=(jax.ShapeDtypeStruct((_NCORES, _NCORES * _NSUB, 2, _CAP),
                                    jnp.int32),
              jax.ShapeDtypeStruct((_NCORES * _NSUB, 16), jnp.int32)),
    mesh=plsc.VectorSubcoreMesh(core_axis_name="c", subcore_axis_name="s"),
    scratch_types=[
        pltpu.VMEM((2, _SCH), jnp.int32),
        pltpu.VMEM((_CAP,), jnp.int32),
        pltpu.VMEM((_CAP,), jnp.int32),
        pltpu.VMEM((_CAP,), jnp.int32),
        pltpu.VMEM((_CAP,), jnp.int32),
        pltpu.VMEM((16,), jnp.int32),
    ],
    compiler_params=_SC_PARAMS_NL,
)


def _edge_body(kd_hbm, qv_hbm, sd_hbm, cnts_hbm, out_hbm,
               sdA, sdB, idxA, idxB, cnt_b,
               krowsA, qvrowsA, krowsB, qvrowsB, acc, agg_sh,
               semA, semB):
    c = lax.axis_index("c")
    s = lax.axis_index("s")
    base_c = c * _HALF

    _zero_vmem(acc, _CH, _H)
    def zc(k, _):
        off = pl.multiple_of(s * _STRIPE + k * _CH, 8)
        pltpu.sync_copy(acc, agg_sh.at[pl.ds(off, _CH)])
        return 0
    lax.fori_loop(0, _STRIPE // _CH, zc, 0)
    zo = pl.multiple_of(s * _STRIPE + (_STRIPE // _CH) * _CH, 8)
    pltpu.sync_copy(acc.at[pl.ds(0, _STRIPE - (_STRIPE // _CH) * _CH)],
                    agg_sh.at[pl.ds(zo, _STRIPE - (_STRIPE // _CH) * _CH)])
    plsc.subcore_barrier()

    def compute_ix(sd, iv):
        def ix(r, _):
            o = pl.multiple_of(r * 16, 16)
            d = sd[1, pl.ds(o, 16)]
            ld = d - base_c
            m = (ld >= 0) & (ld < _HALF)
            iv[pl.ds(o, 16)] = jnp.where(m, ld, _HALF)
            return 0
        lax.fori_loop(0, _CH // 16, ix, 0)

    def gate(krows, qvrows):
        @plsc.parallel_loop(0, _CH, unroll=4)
        def _(e):
            for j in range(_H // 32):
                k32 = krows[e, pl.ds(32 * j, 32)]
                q32 = qvrows[e, pl.ds(32 * j, 32)]
                v32 = qvrows[e, pl.ds(_H + 32 * j, 32)]
                ka, kb = plsc.unpack(k32, format=plsc.PackFormat.INTERLEAVED)
                qa, qb = plsc.unpack(q32, format=plsc.PackFormat.INTERLEAVED)
                va, vb = plsc.unpack(v32, format=plsc.PackFormat.INTERLEAVED)
                ga = 1.0 / (1.0 + jnp.exp(-(ka + qa)))
                gb = 1.0 / (1.0 + jnp.exp(-(kb + qb)))
                acc[e, pl.ds(32 * j, 16)] = ga * va
                acc[e, pl.ds(32 * j + 16, 16)] = gb * vb

    lane = lax.broadcasted_iota(jnp.int32, (16,), 0)
    for sti in range(2):
        st = 2 * s + sti
        pltpu.sync_copy(cnts_hbm.at[st], cnt_b)
        nch = jnp.sum(jnp.where(lane == c, cnt_b[pl.ds(0, 16)], 0))

        def load_fire(ci, sd, kr, qr, sem):
            off = pl.ds(pl.multiple_of(ci * _CH, 8), _CH)
            pltpu.sync_copy(sd_hbm.at[c, st, pl.ds(0, 2), off], sd)
            return (pltpu.async_copy(kd_hbm.at[sd.at[1]], kr, sem),
                    pltpu.async_copy(qv_hbm.at[sd.at[0]], qr, sem))

        def pairf(p, _):
            ca1, ca2 = load_fire(2 * p, sdA, krowsA, qvrowsA, semA)
            cb1, cb2 = load_fire(2 * p + 1, sdB, krowsB, qvrowsB, semB)
            ca1.wait()
            ca2.wait()
            compute_ix(sdA, idxA)
            gate(krowsA, qvrowsA)
            cb1.wait()
            cb2.wait()
            pltpu.sync_copy(acc, agg_sh.at[idxA], add=True)
            compute_ix(sdB, idxB)
            gate(krowsB, qvrowsB)
            pltpu.sync_copy(acc, agg_sh.at[idxB], add=True)
            return 0
        lax.fori_loop(0, nch // 2, pairf, 0)

        @pl.when(lax.rem(nch, 2) == 1)
        def _():
            ca1, ca2 = load_fire(nch - 1, sdA, krowsA, qvrowsA, semA)
            ca1.wait()
            ca2.wait()
            compute_ix(sdA, idxA)
            gate(krowsA, qvrowsA)
            pltpu.sync_copy(acc, agg_sh.at[idxA], add=True)
    plsc.subcore_barrier()

    tail = _HALF - (_NSUB - 1) * _STRIPE
    so = pl.multiple_of(s * _STRIPE, 8)
    pltpu.sync_copy(agg_sh.at[pl.ds(so, tail)],
                    out_hbm.at[pl.ds(base_c + so, tail)])
    @pl.when(s < _NSUB - 1)
    def _():
        o2 = pl.multiple_of(s * _STRIPE + tail, 8)
        pltpu.sync_copy(agg_sh.at[pl.ds(o2, _STRIPE - tail)],
                        out_hbm.at[pl.ds(base_c + o2, _STRIPE - tail)])


_edge_call = functools.partial(
    pl.kernel,
    _edge_body,
    out_type=jax.ShapeDtypeStruct((_N, _H), jnp.float32),
    mesh=plsc.VectorSubcoreMesh(core_axis_name="c", subcore_axis_name="s"),
    scratch_types=[
        pltpu.VMEM((2, _CH), jnp.int32),
        pltpu.VMEM((2, _CH), jnp.int32),
        pltpu.VMEM((_CH,), jnp.int32),
        pltpu.VMEM((_CH,), jnp.int32),
        pltpu.VMEM((16,), jnp.int32),
        pltpu.VMEM((_CH, _H), jnp.bfloat16),
        pltpu.VMEM((_CH, 2 * _H), jnp.bfloat16),
        pltpu.VMEM((_CH, _H), jnp.bfloat16),
        pltpu.VMEM((_CH, 2 * _H), jnp.bfloat16),
        pltpu.VMEM((_CH, _H), jnp.float32),
        pltpu.VMEM_SHARED((_AGG_ROWS, _H), jnp.float32),
        pltpu.SemaphoreType.DMA,
        pltpu.SemaphoreType.DMA,
    ],
    compiler_params=_SC_PARAMS_NL,
)


def _pool_body(agg_hbm, sk_hbm, batch_hbm, sums_hbm, cnts_hbm,
               b_v, rows_a, rows_s, ones_v, zbuf2, sums_sh, cnts_sh):
    c = lax.axis_index("c")
    s = lax.axis_index("s")
    wid = c * _NSUB + s

    _zero_vmem(rows_a, 32, _H)
    _zero_vmem(zbuf2, 32, 16)
    def fill_ones(r, _):
        ones_v[r, pl.ds(0, 16)] = jnp.full((16,), 1.0, jnp.float32)
        return 0
    lax.fori_loop(0, _PCH, fill_ones, 0)
    so = pl.multiple_of(s * 32, 8)
    pltpu.sync_copy(rows_a.at[pl.ds(0, 32)], sums_sh.at[pl.ds(so, 32)])
    pltpu.sync_copy(zbuf2, cnts_sh.at[pl.ds(so, 32)])
    plsc.subcore_barrier()

    def step(k, _):
        ch = k * (_NCORES * _NSUB) + wid
        @pl.when(ch < _PNCH)
        def _():
            off = pl.multiple_of(ch * _PCH, 8)
            pltpu.sync_copy(agg_hbm.at[pl.ds(off, _PCH)], rows_a)
            pltpu.sync_copy(sk_hbm.at[pl.ds(off, _PCH)], rows_s)
            pltpu.sync_copy(batch_hbm.at[pl.ds(off, _PCH)], b_v)
            def addrow(e, _):
                for j in range(_H // 16):
                    rows_a[e, pl.ds(16 * j, 16)] = (
                        rows_a[e, pl.ds(16 * j, 16)] + rows_s[e, pl.ds(16 * j, 16)])
                return 0
            lax.fori_loop(0, _PCH, addrow, 0)
            pltpu.sync_copy(rows_a, sums_sh.at[b_v], add=True)
            pltpu.sync_copy(ones_v, cnts_sh.at[b_v], add=True)
        return 0
    lax.fori_loop(0, _PK, step, 0)
    plsc.subcore_barrier()

    @pl.when(s == 0)
    def _():
        pltpu.sync_copy(sums_sh, sums_hbm.at[c])
        pltpu.sync_copy(cnts_sh, cnts_hbm.at[c])


_pool_call = functools.partial(
    pl.kernel,
    _pool_body,
    out_type=(jax.ShapeDtypeStruct((_NCORES, _G, _H), jnp.float32),
              jax.ShapeDtypeStruct((_NCORES, _G, 16), jnp.float32)),
    mesh=plsc.VectorSubcoreMesh(core_axis_name="c", subcore_axis_name="s"),
    scratch_types=[
        pltpu.VMEM((_PCH,), jnp.int32),
        pltpu.VMEM((_PCH, _H), jnp.float32),
        pltpu.VMEM((_PCH, _H), jnp.float32),
        pltpu.VMEM((_PCH, 16), jnp.float32),
        pltpu.VMEM((32, 16), jnp.float32),
        pltpu.VMEM_SHARED((_G, _H), jnp.float32),
        pltpu.VMEM_SHARED((_G, 16), jnp.float32),
    ],
    compiler_params=_SC_PARAMS,
)


_RB = 400
_NB = _N // _RB


def _dense1_body(x_ref, wk, bk, wqv, bqv, ws, bs, kd, qv, sk):
    X = x_ref[...]
    kd[...] = (X * wk[...] + bk[...]).astype(jnp.bfloat16)
    qv[...] = (X * wqv[...] + bqv[...]).astype(jnp.bfloat16)
    sk[...] = X * ws[...] + bs[...]


def _dense2_body(agg_ref, skin_ref, wk, bk, wqv, bqv, ws, bs, kd, qv, sk):
    X = jnp.maximum(agg_ref[...] + skin_ref[...], 0.0)
    kd[...] = (jnp.dot(X, wk[...], preferred_element_type=jnp.float32)
               + bk[...]).astype(jnp.bfloat16)
    qv[...] = (jnp.dot(X, wqv[...], preferred_element_type=jnp.float32)
               + bqv[...]).astype(jnp.bfloat16)
    sk[...] = jnp.dot(X, ws[...], preferred_element_type=jnp.float32) + bs[...]


def _wspec(din, dout):
    return pl.BlockSpec((din, dout), lambda i: (0, 0))


def _nspec(width):
    return pl.BlockSpec((_RB, width), lambda i: (i, 0))


def _dense_out_types():
    return (jax.ShapeDtypeStruct((_N, _H), jnp.bfloat16),
            jax.ShapeDtypeStruct((_N, 2 * _H), jnp.bfloat16),
            jax.ShapeDtypeStruct((_N, _H), jnp.float32))


def _dense1(x, wk, bk, wqv, bqv, ws, bs):
    return pl.pallas_call(
        _dense1_body,
        grid=(_NB,),
        in_specs=[_nspec(1), _wspec(1, _H), _wspec(1, _H), _wspec(1, 2 * _H),
                  _wspec(1, 2 * _H), _wspec(1, _H), _wspec(1, _H)],
        out_specs=[_nspec(_H), _nspec(2 * _H), _nspec(_H)],
        out_shape=_dense_out_types(),
    )(x, wk, bk, wqv, bqv, ws, bs)


def _dense2(agg, skin, wk, bk, wqv, bqv, ws, bs):
    return pl.pallas_call(
        _dense2_body,
        grid=(_NB,),
        in_specs=[_nspec(_H), _nspec(_H), _wspec(_H, _H), _wspec(1, _H),
                  _wspec(_H, 2 * _H), _wspec(1, 2 * _H), _wspec(_H, _H),
                  _wspec(1, _H)],
        out_specs=[_nspec(_H), _nspec(2 * _H), _nspec(_H)],
        out_shape=_dense_out_types(),
    )(agg, skin, wk, bk, wqv, bqv, ws, bs)


def _final_body(sums_ref, cnts_ref, w, b, out_ref):
    sm = sums_ref[0] + sums_ref[1]
    cn = cnts_ref[0] + cnts_ref[1]
    cnt = cn[:, 0:1]
    pooled = sm / jnp.maximum(cnt, 1.0)
    out_ref[...] = jnp.dot(pooled, w[...], preferred_element_type=jnp.float32) + b[...]


def _final(sums, cnts, w, b):
    return pl.pallas_call(
        _final_body,
        grid=(1,),
        in_specs=[pl.BlockSpec((_NCORES, _G, _H), lambda i: (0, 0, 0)),
                  pl.BlockSpec((_NCORES, _G, 16), lambda i: (0, 0, 0)),
                  _wspec(_H, _C), _wspec(1, _C)],
        out_specs=pl.BlockSpec((_G, _C), lambda i: (0, 0)),
        out_shape=jax.ShapeDtypeStruct((_G, _C), jnp.float32),
    )(sums, cnts, w, b)


def _conv_weights(p, permute_in):
    up = jnp.array(_UPERM, jnp.int32)
    def rows(w):
        return w[up, :] if permute_in else w
    wk = rows(p["key"]["W"])
    bk = p["key"]["b"].reshape(1, -1)
    wqv = jnp.concatenate([rows(p["query"]["W"]), rows(p["value"]["W"])], axis=1)
    bqv = jnp.concatenate([p["query"]["b"], p["value"]["b"]]).reshape(1, -1)
    ws = rows(p["skip"]["W"])[:, up]
    bs = p["skip"]["b"][up].reshape(1, -1)
    return wk, bk, wqv, bqv, ws, bs


def kernel(x, edge_index, batch, params):
    pad_e = jnp.concatenate(
        [edge_index,
         jnp.concatenate([jnp.zeros((1, _E2 - _E), jnp.int32),
                          jnp.full((1, _E2 - _E), _N, jnp.int32)])], axis=1)
    e3 = pad_e.reshape(2, _E2 // _SCH, _SCH).transpose(1, 0, 2)
    sd, cnts = _part_call()(e3)

    kd, qv, sk = _dense1(x, *_conv_weights(params["conv1"], False))
    agg = _edge_call()(kd, qv, sd, cnts)
    for name in ("conv2", "conv2_1", "conv3"):
        kd, qv, sk = _dense2(agg, sk, *_conv_weights(params[name], True))
        agg = _edge_call()(kd, qv, sd, cnts)

    sums, pcnts = _pool_call()(agg, sk, batch)
    up = jnp.array(_UPERM, jnp.int32)
    return _final(sums, pcnts, params["lin"]["W"][up, :],
                  params["lin"]["b"].reshape(1, -1))


# speedup vs baseline: 7.7250x; 1.0910x over previous
import functools

import jax
import jax.numpy as jnp
from jax import lax
from jax.experimental import pallas as pl
from jax.experimental.pallas import tpu as pltpu
import jax.experimental.pallas.tpu_sc as plsc

_N = 50000
_E = 800000
_H = 64
_G = 512
_C = 10

_NCORES = 2
_NSUB = 16
_HALF = _N // _NCORES
_STRIPE = 1568
_AGG_ROWS = _NSUB * _STRIPE
_CH = 64
_E2 = 800256
_SCH = 48
_SSEG = _E2 // (_NCORES * _NSUB * _SCH)
_CAP = 25088

_PCH = 80
_PNCH = _N // _PCH
_PK = -(-_PNCH // (_NCORES * _NSUB))

_UPERM = sum((([32 * b + i for i in range(0, 32, 2)] +
               [32 * b + i for i in range(1, 32, 2)]) for b in (0, 1)), [])

_SC_PARAMS = pltpu.CompilerParams(use_tc_tiling_on_sc=False)
_SC_PARAMS_NL = pltpu.CompilerParams(use_tc_tiling_on_sc=False,
                                     needs_layout_passes=False)


def _zero_vmem(ref, rows, width):
    def body(r, _):
        for j in range(width // 16):
            ref[r, pl.ds(16 * j, 16)] = jnp.zeros((16,), jnp.float32)
        return 0
    lax.fori_loop(0, rows, body, 0)


def _part_body(e3_hbm, sd_hbm, cnts_hbm,
               chunk_v, b0s, b0d, b1s, b1d, cntv):
    c = lax.axis_index("c")
    s = lax.axis_index("s")
    t = c * _NSUB + s
    lane16 = lax.broadcasted_iota(jnp.int32, (16,), 0)

    def scan_chunk(ci, carry):
        pltpu.sync_copy(e3_hbm.at[t * _SSEG + ci], chunk_v)
        c0, c1 = carry
        for g in range(_SCH // 16):
            sv = chunk_v[0, pl.ds(16 * g, 16)]
            dv = chunk_v[1, pl.ds(16 * g, 16)]
            m0 = dv < _HALF
            mi = jnp.where(m0, 1, 0)
            pos0 = c0 + plsc.cumsum(mi) - 1
            pos1 = c1 + plsc.cumsum(1 - mi) - 1
            plsc.store_scatter(b0s, [pos0], sv, mask=m0)
            plsc.store_scatter(b0d, [pos0], dv, mask=m0)
            m1 = jnp.logical_not(m0)
            plsc.store_scatter(b1s, [pos1], sv, mask=m1)
            plsc.store_scatter(b1d, [pos1], dv, mask=m1)
            n0 = jnp.sum(mi)
            c0 = c0 + n0
            c1 = c1 + (16 - n0)
        return (c0, c1)
    cnt0, cnt1 = lax.fori_loop(0, _SSEG, scan_chunk,
                               (jnp.int32(0), jnp.int32(0)))

    zero16 = jnp.zeros((16,), jnp.int32)
    garb16 = jnp.full((16,), _N, jnp.int32)
    for k in range(_CH // 16):
        plsc.store_scatter(b0s, [cnt0 + lane16 + 16 * k], zero16)
        plsc.store_scatter(b0d, [cnt0 + lane16 + 16 * k], garb16)
        plsc.store_scatter(b1s, [cnt1 + lane16 + 16 * k], zero16)
        plsc.store_scatter(b1d, [cnt1 + lane16 + 16 * k], garb16)
    c0p = ((cnt0 + _CH - 1) // _CH) * _CH
    c1p = ((cnt1 + _CH - 1) // _CH) * _CH

    pltpu.sync_copy(b0s, sd_hbm.at[0, t, 0])
    pltpu.sync_copy(b0d, sd_hbm.at[0, t, 1])
    pltpu.sync_copy(b1s, sd_hbm.at[1, t, 0])
    pltpu.sync_copy(b1d, sd_hbm.at[1, t, 1])
    cntv[pl.ds(0, 16)] = jnp.where(lane16 == 0, c0p // _CH,
                                   jnp.where(lane16 == 1, c1p // _CH, 0))
    pltpu.sync_copy(cntv, cnts_hbm.at[t])


_part_call = functools.partial(
    pl.kernel,
    _part_body,
    out_type=(jax.ShapeDtypeStruct((_NCORES, _NCORES * _NSUB, 2, _CAP),
                                    jnp.int32),
              jax.ShapeDtypeStruct((_NCORES * _NSUB, 16), jnp.int32)),
    mesh=plsc.VectorSubcoreMesh(core_axis_name="c", subcore_axis_name="s"),
    scratch_types=[
        pltpu.VMEM((2, _SCH), jnp.int32),
        pltpu.VMEM((_CAP,), jnp.int32),
        pltpu.VMEM((_CAP,), jnp.int32),
        pltpu.VMEM((_CAP,), jnp.int32),
        pltpu.VMEM((_CAP,), jnp.int32),
        pltpu.VMEM((16,), jnp.int32),
    ],
    compiler_params=_SC_PARAMS_NL,
)


def _edge_body(kd_hbm, qv_hbm, sd_hbm, cnts_hbm, out_hbm,
               sdA, sdB, idxA, idxB, cnt_b,
               krowsA, qvrowsA, krowsB, qvrowsB, acc, agg_sh,
               semA, semB):
    c = lax.axis_index("c")
    s = lax.axis_index("s")
    base_c = c * _HALF

    _zero_vmem(acc, _CH, _H)
    def zc(k, _):
        off = pl.multiple_of(s * _STRIPE + k * _CH, 8)
        pltpu.sync_copy(acc, agg_sh.at[pl.ds(off, _CH)])
        return 0
    lax.fori_loop(0, _STRIPE // _CH, zc, 0)
    zo = pl.multiple_of(s * _STRIPE + (_STRIPE // _CH) * _CH, 8)
    pltpu.sync_copy(acc.at[pl.ds(0, _STRIPE - (_STRIPE // _CH) * _CH)],
                    agg_sh.at[pl.ds(zo, _STRIPE - (_STRIPE // _CH) * _CH)])
    plsc.subcore_barrier()

    def compute_ix(sd, iv):
        def ix(r, _):
            o = pl.multiple_of(r * 16, 16)
            d = sd[1, pl.ds(o, 16)]
            ld = d - base_c
            m = (ld >= 0) & (ld < _HALF)
            iv[pl.ds(o, 16)] = jnp.where(m, ld, _HALF)
            return 0
        lax.fori_loop(0, _CH // 16, ix, 0)

    def gate(krows, qvrows):
        @plsc.parallel_loop(0, _CH, unroll=4)
        def _(e):
            for j in range(_H // 32):
                k32 = krows[e, pl.ds(32 * j, 32)]
                q32 = qvrows[e, pl.ds(32 * j, 32)]
                v32 = qvrows[e, pl.ds(_H + 32 * j, 32)]
                ka, kb = plsc.unpack(k32, format=plsc.PackFormat.INTERLEAVED)
                qa, qb = plsc.unpack(q32, format=plsc.PackFormat.INTERLEAVED)
                va, vb = plsc.unpack(v32, format=plsc.PackFormat.INTERLEAVED)
                ga = 1.0 / (1.0 + jnp.exp(-(ka + qa)))
                gb = 1.0 / (1.0 + jnp.exp(-(kb + qb)))
                acc[e, pl.ds(32 * j, 16)] = ga * va
                acc[e, pl.ds(32 * j + 16, 16)] = gb * vb

    lane = lax.broadcasted_iota(jnp.int32, (16,), 0)
    for sti in range(2):
        st = 2 * s + sti
        pltpu.sync_copy(cnts_hbm.at[st], cnt_b)
        nch = jnp.sum(jnp.where(lane == c, cnt_b[pl.ds(0, 16)], 0))

        def load_fire(ci, sd, kr, qr, sem):
            off = pl.ds(pl.multiple_of(ci * _CH, 8), _CH)
            pltpu.sync_copy(sd_hbm.at[c, st, pl.ds(0, 2), off], sd)
            return (pltpu.async_copy(kd_hbm.at[sd.at[1]], kr, sem),
                    pltpu.async_copy(qv_hbm.at[sd.at[0]], qr, sem))

        def pairf(p, _):
            ca1, ca2 = load_fire(2 * p, sdA, krowsA, qvrowsA, semA)
            cb1, cb2 = load_fire(2 * p + 1, sdB, krowsB, qvrowsB, semB)
            ca1.wait()
            ca2.wait()
            compute_ix(sdA, idxA)
            gate(krowsA, qvrowsA)
            cb1.wait()
            cb2.wait()
            pltpu.sync_copy(acc, agg_sh.at[idxA], add=True)
            compute_ix(sdB, idxB)
            gate(krowsB, qvrowsB)
            pltpu.sync_copy(acc, agg_sh.at[idxB], add=True)
            return 0
        lax.fori_loop(0, nch // 2, pairf, 0)

        @pl.when(lax.rem(nch, 2) == 1)
        def _():
            ca1, ca2 = load_fire(nch - 1, sdA, krowsA, qvrowsA, semA)
            ca1.wait()
            ca2.wait()
            compute_ix(sdA, idxA)
            gate(krowsA, qvrowsA)
            pltpu.sync_copy(acc, agg_sh.at[idxA], add=True)
    plsc.subcore_barrier()

    tail = _HALF - (_NSUB - 1) * _STRIPE
    so = pl.multiple_of(s * _STRIPE, 8)
    pltpu.sync_copy(agg_sh.at[pl.ds(so, tail)],
                    out_hbm.at[pl.ds(base_c + so, tail)])
    @pl.when(s < _NSUB - 1)
    def _():
        o2 = pl.multiple_of(s * _STRIPE + tail, 8)
        pltpu.sync_copy(agg_sh.at[pl.ds(o2, _STRIPE - tail)],
                        out_hbm.at[pl.ds(base_c + o2, _STRIPE - tail)])


_edge_call = functools.partial(
    pl.kernel,
    _edge_body,
    out_type=jax.ShapeDtypeStruct((_N, _H), jnp.float32),
    mesh=plsc.VectorSubcoreMesh(core_axis_name="c", subcore_axis_name="s"),
    scratch_types=[
        pltpu.VMEM((2, _CH), jnp.int32),
        pltpu.VMEM((2, _CH), jnp.int32),
        pltpu.VMEM((_CH,), jnp.int32),
        pltpu.VMEM((_CH,), jnp.int32),
        pltpu.VMEM((16,), jnp.int32),
        pltpu.VMEM((_CH, _H), jnp.bfloat16),
        pltpu.VMEM((_CH, 2 * _H), jnp.bfloat16),
        pltpu.VMEM((_CH, _H), jnp.bfloat16),
        pltpu.VMEM((_CH, 2 * _H), jnp.bfloat16),
        pltpu.VMEM((_CH, _H), jnp.float32),
        pltpu.VMEM_SHARED((_AGG_ROWS, _H), jnp.float32),
        pltpu.SemaphoreType.DMA,
        pltpu.SemaphoreType.DMA,
    ],
    compiler_params=_SC_PARAMS_NL,
)


def _pool_body(agg_hbm, sk_hbm, batch_hbm, sums_hbm, cnts_hbm,
               b_v, rows_a, rows_s, ones_v, zbuf2, sums_sh, cnts_sh):
    c = lax.axis_index("c")
    s = lax.axis_index("s")
    wid = c * _NSUB + s

    _zero_vmem(rows_a, 32, _H)
    _zero_vmem(zbuf2, 32, 16)
    def fill_ones(r, _):
        ones_v[r, pl.ds(0, 16)] = jnp.full((16,), 1.0, jnp.float32)
        return 0
    lax.fori_loop(0, _PCH, fill_ones, 0)
    so = pl.multiple_of(s * 32, 8)
    pltpu.sync_copy(rows_a.at[pl.ds(0, 32)], sums_sh.at[pl.ds(so, 32)])
    pltpu.sync_copy(zbuf2, cnts_sh.at[pl.ds(so, 32)])
    plsc.subcore_barrier()

    def step(k, _):
        ch = k * (_NCORES * _NSUB) + wid
        @pl.when(ch < _PNCH)
        def _():
            off = pl.multiple_of(ch * _PCH, 8)
            pltpu.sync_copy(agg_hbm.at[pl.ds(off, _PCH)], rows_a)
            pltpu.sync_copy(sk_hbm.at[pl.ds(off, _PCH)], rows_s)
            pltpu.sync_copy(batch_hbm.at[pl.ds(off, _PCH)], b_v)
            def addrow(e, _):
                for j in range(_H // 16):
                    rows_a[e, pl.ds(16 * j, 16)] = (
                        rows_a[e, pl.ds(16 * j, 16)] + rows_s[e, pl.ds(16 * j, 16)])
                return 0
            lax.fori_loop(0, _PCH, addrow, 0)
            pltpu.sync_copy(rows_a, sums_sh.at[b_v], add=True)
            pltpu.sync_copy(ones_v, cnts_sh.at[b_v], add=True)
        return 0
    lax.fori_loop(0, _PK, step, 0)
    plsc.subcore_barrier()

    @pl.when(s == 0)
    def _():
        pltpu.sync_copy(sums_sh, sums_hbm.at[c])
        pltpu.sync_copy(cnts_sh, cnts_hbm.at[c])


_pool_call = functools.partial(
    pl.kernel,
    _pool_body,
    out_type=(jax.ShapeDtypeStruct((_NCORES, _G, _H), jnp.float32),
              jax.ShapeDtypeStruct((_NCORES, _G, 16), jnp.float32)),
    mesh=plsc.VectorSubcoreMesh(core_axis_name="c", subcore_axis_name="s"),
    scratch_types=[
        pltpu.VMEM((_PCH,), jnp.int32),
        pltpu.VMEM((_PCH, _H), jnp.float32),
        pltpu.VMEM((_PCH, _H), jnp.float32),
        pltpu.VMEM((_PCH, 16), jnp.float32),
        pltpu.VMEM((32, 16), jnp.float32),
        pltpu.VMEM_SHARED((_G, _H), jnp.float32),
        pltpu.VMEM_SHARED((_G, 16), jnp.float32),
    ],
    compiler_params=_SC_PARAMS,
)


_RB = 400
_NB = _N // _RB


def _dense1_body(x_ref, wk, bk, wqv, bqv, ws, bs, kd, qv, sk):
    X = x_ref[...]
    kd[...] = (X * wk[...] + bk[...]).astype(jnp.bfloat16)
    qv[...] = (X * wqv[...] + bqv[...]).astype(jnp.bfloat16)
    sk[...] = X * ws[...] + bs[...]


def _dense2_body(agg_ref, skin_ref, wk, bk, wqv, bqv, ws, bs, kd, qv, sk):
    X = jnp.maximum(agg_ref[...] + skin_ref[...], 0.0)
    kd[...] = (jnp.dot(X, wk[...], preferred_element_type=jnp.float32)
               + bk[...]).astype(jnp.bfloat16)
    qv[...] = (jnp.dot(X, wqv[...], preferred_element_type=jnp.float32)
               + bqv[...]).astype(jnp.bfloat16)
    sk[...] = jnp.dot(X, ws[...], preferred_element_type=jnp.float32) + bs[...]


def _wspec(din, dout):
    return pl.BlockSpec((din, dout), lambda i: (0, 0))


def _nspec(width):
    return pl.BlockSpec((_RB, width), lambda i: (i, 0))


def _dense_out_types():
    return (jax.ShapeDtypeStruct((_N, _H), jnp.bfloat16),
            jax.ShapeDtypeStruct((_N, 2 * _H), jnp.bfloat16),
            jax.ShapeDtypeStruct((_N, _H), jnp.float32))


def _dense1(x, wk, bk, wqv, bqv, ws, bs):
    return pl.pallas_call(
        _dense1_body,
        grid=(_NB,),
        in_specs=[_nspec(1), _wspec(1, _H), _wspec(1, _H), _wspec(1, 2 * _H),
                  _wspec(1, 2 * _H), _wspec(1, _H), _wspec(1, _H)],
        out_specs=[_nspec(_H), _nspec(2 * _H), _nspec(_H)],
        out_shape=_dense_out_types(),
    )(x, wk, bk, wqv, bqv, ws, bs)


def _dense2(agg, skin, wk, bk, wqv, bqv, ws, bs):
    return pl.pallas_call(
        _dense2_body,
        grid=(_NB,),
        in_specs=[_nspec(_H), _nspec(_H), _wspec(_H, _H), _wspec(1, _H),
                  _wspec(_H, 2 * _H), _wspec(1, 2 * _H), _wspec(_H, _H),
                  _wspec(1, _H)],
        out_specs=[_nspec(_H), _nspec(2 * _H), _nspec(_H)],
        out_shape=_dense_out_types(),
    )(agg, skin, wk, bk, wqv, bqv, ws, bs)


def _final_body(sums_ref, cnts_ref, w, b, out_ref):
    sm = sums_ref[0] + sums_ref[1]
    cn = cnts_ref[0] + cnts_ref[1]
    cnt = cn[:, 0:1]
    pooled = sm / jnp.maximum(cnt, 1.0)
    out_ref[...] = jnp.dot(pooled, w[...], preferred_element_type=jnp.float32) + b[...]


def _final(sums, cnts, w, b):
    return pl.pallas_call(
        _final_body,
        grid=(1,),
        in_specs=[pl.BlockSpec((_NCORES, _G, _H), lambda i: (0, 0, 0)),
                  pl.BlockSpec((_NCORES, _G, 16), lambda i: (0, 0, 0)),
                  _wspec(_H, _C), _wspec(1, _C)],
        out_specs=pl.BlockSpec((_G, _C), lambda i: (0, 0)),
        out_shape=jax.ShapeDtypeStruct((_G, _C), jnp.float32),
    )(sums, cnts, w, b)


def _conv_weights(p, permute_in):
    up = jnp.array(_UPERM, jnp.int32)
    def rows(w):
        return w[up, :] if permute_in else w
    wk = rows(p["key"]["W"])
    bk = p["key"]["b"].reshape(1, -1)
    wqv = jnp.concatenate([rows(p["query"]["W"]), rows(p["value"]["W"])], axis=1)
    bqv = jnp.concatenate([p["query"]["b"], p["value"]["b"]]).reshape(1, -1)
    ws = rows(p["skip"]["W"])[:, up]
    bs = p["skip"]["b"][up].reshape(1, -1)
    return wk, bk, wqv, bqv, ws, bs


def kernel(x, edge_index, batch, params):
    pad_e = jnp.concatenate(
        [edge_index,
         jnp.concatenate([jnp.zeros((1, _E2 - _E), jnp.int32),
                          jnp.full((1, _E2 - _E), _N, jnp.int32)])], axis=1)
    e3 = pad_e.reshape(2, _E2 // _SCH, _SCH).transpose(1, 0, 2)
    sd, cnts = _part_call()(e3)

    kd, qv, sk = _dense1(x, *_conv_weights(params["conv1"], False))
    agg = _edge_call()(kd, qv, sd, cnts)
    for name in ("conv2", "conv2_1", "conv3"):
        kd, qv, sk = _dense2(agg, sk, *_conv_weights(params[name], True))
        agg = _edge_call()(kd, qv, sd, cnts)

    sums, pcnts = _pool_call()(agg, sk, batch)
    up = jnp.array(_UPERM, jnp.int32)
    return _final(sums, pcnts, params["lin"]["W"][up, :],
                  params["lin"]["b"].reshape(1, -1))
